# baseline (speedup 1.0000x reference)
import functools

import jax
import jax.numpy as jnp
from jax import lax
from jax.experimental import pallas as pl
from jax.experimental.pallas import tpu as pltpu
from jax.experimental.pallas import tpu_sc as plsc

N = 50000
E = 800000
EL = 200000
NC, NS = 2, 16
NW = NC * NS
CHUNK = 256
KE = 100
EP = NW * KE * CHUNK
IB = 10
KE0, KE1 = 130, 70
KL = 28
ELP = NW * KL * CHUNK
KL0, KL1 = 36, 20
LB = 4
ACC_ROWS = 50176
APT = ACC_ROWS // NS
ZC = 112


def _sc_mesh():
    return plsc.VectorSubcoreMesh(
        core_axis_name="c", subcore_axis_name="s",
        num_cores=NC, num_subcores=NS)


_SC_PARAMS = pltpu.CompilerParams(use_tc_tiling_on_sc=False)


def _deg_kernel(dst2d, ones, zrows):
    @functools.partial(
        pl.kernel,
        out_type=jax.ShapeDtypeStruct((NC, ACC_ROWS), jnp.float32),
        mesh=_sc_mesh(),
        compiler_params=_SC_PARAMS,
        scratch_types=[
            pltpu.VMEM((KE, CHUNK), jnp.int32),
            pltpu.VMEM((CHUNK,), jnp.float32),
            pltpu.VMEM((APT,), jnp.float32),
            pltpu.VMEM_SHARED((ACC_ROWS,), jnp.float32),
        ],
    )
    def deg(dst_hbm, ones_hbm, z_hbm, out_hbm, idx_v, ones_v, zbuf, acc):
        c = lax.axis_index("c")
        s = lax.axis_index("s")
        wid = c * NS + s
        sl = pl.ds(s * APT, APT)
        pltpu.sync_copy(z_hbm, zbuf)
        pltpu.sync_copy(zbuf, acc.at[sl])
        pltpu.sync_copy(ones_hbm, ones_v)
        pltpu.sync_copy(dst_hbm.at[pl.ds(wid * KE, KE)], idx_v)
        plsc.subcore_barrier()

        def body(j, carry):
            pltpu.sync_copy(ones_v, acc.at[idx_v.at[j]], add=True)
            return carry
        lax.fori_loop(0, KE, body, 0)
        plsc.subcore_barrier()
        pltpu.sync_copy(acc.at[sl], zbuf)
        pltpu.sync_copy(zbuf, out_hbm.at[c, sl])

    return deg(dst2d, ones, zrows)


def _conv_agg(t3, src2d, dst2d, zrows, F):
    @functools.partial(
        pl.kernel,
        out_type=jax.ShapeDtypeStruct((NC, ACC_ROWS, F), jnp.float32),
        mesh=_sc_mesh(),
        compiler_params=_SC_PARAMS,
        scratch_types=[
            pltpu.VMEM((IB, CHUNK), jnp.int32),
            pltpu.VMEM((IB, CHUNK), jnp.int32),
            pltpu.VMEM((CHUNK, F), jnp.float32),
            pltpu.VMEM((CHUNK, F), jnp.float32),
            pltpu.VMEM((ZC, F), jnp.float32),
            pltpu.SemaphoreType.DMA,
            pltpu.SemaphoreType.DMA,
            pltpu.SemaphoreType.DMA,
            pltpu.SemaphoreType.DMA,
            pltpu.VMEM_SHARED((ACC_ROWS, F), jnp.float32),
        ],
    )
    def agg(t3_hbm, src_hbm, dst_hbm, z_hbm, out_hbm,
            src_v, dst_v, rows0, rows1, zstage, sg0, sg1, ss0, ss1, acc):
        c = lax.axis_index("c")
        s = lax.axis_index("s")
        wid = c * NS + s
        tbl = t3_hbm.at[c]
        rows = (rows0, rows1)
        sg = (sg0, sg1)
        ss = (ss0, ss1)
        pltpu.sync_copy(z_hbm, zstage)

        def zbody(k, carry):
            pltpu.sync_copy(zstage, acc.at[pl.ds(s * APT + k * ZC, ZC)])
            return carry
        lax.fori_loop(0, APT // ZC, zbody, 0)
        plsc.subcore_barrier()

        base = jnp.where(c == 0, s * KE0, NS * KE0 + s * KE1)
        ng = jnp.where(c == 0, KE0 // IB, KE1 // IB)

        def outer(g, carry):
            gb = base + g * IB
            pltpu.sync_copy(src_hbm.at[pl.ds(gb, IB)], src_v)
            pltpu.sync_copy(dst_hbm.at[pl.ds(gb, IB)], dst_v)
            dg = [None, None]
            dsc = [None, None]
            dg[0] = pltpu.async_copy(tbl.at[src_v.at[0]], rows[0], sg[0])
            for k in range(IB):
                b = k % 2
                nb = 1 - b
                dg[b].wait()
                if k + 1 < IB:
                    if dsc[nb] is not None:
                        dsc[nb].wait()
                    dg[nb] = pltpu.async_copy(
                        tbl.at[src_v.at[k + 1]], rows[nb], sg[nb])
                dsc[b] = pltpu.async_copy(
                    rows[b], acc.at[dst_v.at[k]], ss[b], add=True)
            dsc[0].wait()
            dsc[1].wait()
            return carry
        lax.fori_loop(0, ng, outer, 0)
        plsc.subcore_barrier()

        def obody(k, carry):
            sk = pl.ds(s * APT + k * ZC, ZC)
            pltpu.sync_copy(acc.at[sk], zstage)
            pltpu.sync_copy(zstage, out_hbm.at[c, sk])
            return carry
        lax.fori_loop(0, APT // ZC, obody, 0)

    return agg(t3, src2d, dst2d, zrows)


def _label_gather(h2, lsrc2d, ldst2d):
    KLM = max(KL0, KL1)

    @functools.partial(
        pl.kernel,
        out_type=(jax.ShapeDtypeStruct((ELP, 32), jnp.float32),
                  jax.ShapeDtypeStruct((ELP, 32), jnp.float32)),
        mesh=_sc_mesh(),
        compiler_params=_SC_PARAMS,
        scratch_types=[
            pltpu.VMEM((KLM, CHUNK), jnp.int32),
            pltpu.VMEM((KLM, CHUNK), jnp.int32),
            pltpu.VMEM((CHUNK, 32), jnp.float32),
            pltpu.VMEM((CHUNK, 32), jnp.float32),
            pltpu.VMEM((CHUNK, 32), jnp.float32),
            pltpu.VMEM((CHUNK, 32), jnp.float32),
            pltpu.SemaphoreType.DMA,
            pltpu.SemaphoreType.DMA,
            pltpu.SemaphoreType.DMA,
            pltpu.SemaphoreType.DMA,
            pltpu.SemaphoreType.DMA,
            pltpu.SemaphoreType.DMA,
            pltpu.SemaphoreType.DMA,
            pltpu.SemaphoreType.DMA,
        ],
    )
    def lab(h2_hbm, src_hbm, dst_hbm, efs_hbm, efd_hbm, src_v, dst_v,
            bs0, bs1, bd0, bd1, gs0, gs1, gd0, gd1, ws0, ws1, wd0, wd1):
        c = lax.axis_index("c")
        s = lax.axis_index("s")
        base = jnp.where(c == 0, s * KL0, NS * KL0 + s * KL1)
        ng = jnp.where(c == 0, KL0 // LB, KL1 // LB)

        @pl.when(c == 0)
        def _():
            pltpu.sync_copy(src_hbm.at[pl.ds(s * KL0, KL0)],
                            src_v.at[pl.ds(0, KL0)])
            pltpu.sync_copy(dst_hbm.at[pl.ds(s * KL0, KL0)],
                            dst_v.at[pl.ds(0, KL0)])

        @pl.when(c == 1)
        def _():
            pltpu.sync_copy(src_hbm.at[pl.ds(NS * KL0 + s * KL1, KL1)],
                            src_v.at[pl.ds(0, KL1)])
            pltpu.sync_copy(dst_hbm.at[pl.ds(NS * KL0 + s * KL1, KL1)],
                            dst_v.at[pl.ds(0, KL1)])

        bufs_s = (bs0, bs1)
        bufs_d = (bd0, bd1)
        gs = (gs0, gs1)
        gd = (gd0, gd1)
        ws = (ws0, ws1)
        wd = (wd0, wd1)

        def outer(g, carry):
            jb = g * LB
            dgs = [None, None]
            dgd = [None, None]
            dss = [None, None]
            dsd = [None, None]
            dgs[0] = pltpu.async_copy(
                h2_hbm.at[src_v.at[jb]], bufs_s[0], gs[0])
            dgd[0] = pltpu.async_copy(
                h2_hbm.at[dst_v.at[jb]], bufs_d[0], gd[0])
            for k in range(LB):
                b = k % 2
                nb = 1 - b
                j = jb + k
                off = pl.ds((base + j) * CHUNK, CHUNK)
                dgs[b].wait()
                dgd[b].wait()
                if k + 1 < LB:
                    if dss[nb] is not None:
                        dss[nb].wait()
                    if dsd[nb] is not None:
                        dsd[nb].wait()
                    dgs[nb] = pltpu.async_copy(
                        h2_hbm.at[src_v.at[j + 1]], bufs_s[nb], gs[nb])
                    dgd[nb] = pltpu.async_copy(
                        h2_hbm.at[dst_v.at[j + 1]], bufs_d[nb], gd[nb])
                dss[b] = pltpu.async_copy(bufs_s[b], efs_hbm.at[off], ws[b])
                dsd[b] = pltpu.async_copy(bufs_d[b], efd_hbm.at[off], wd[b])
            dss[0].wait()
            dss[1].wait()
            dsd[0].wait()
            dsd[1].wait()
            return carry
        lax.fori_loop(0, ng, outer, 0)

    return lab(h2, lsrc2d, ldst2d)


_R = 1024
_TG = ACC_ROWS // _R


def _tc_encoder(x, W_enc, b_enc, W1):
    def body(x_ref, we_ref, be_ref, w1_ref, o_ref):
        h = jnp.dot(x_ref[...], we_ref[...],
                    preferred_element_type=jnp.float32) + be_ref[...]
        h = jnp.maximum(h, 0.0)
        o_ref[...] = jnp.dot(h, w1_ref[...],
                             preferred_element_type=jnp.float32)
    return pl.pallas_call(
        body,
        grid=(_TG,),
        in_specs=[
            pl.BlockSpec((_R, 128), lambda i: (i, 0)),
            pl.BlockSpec((128, 64), lambda i: (0, 0)),
            pl.BlockSpec((1, 64), lambda i: (0, 0)),
            pl.BlockSpec((64, 64), lambda i: (0, 0)),
        ],
        out_specs=pl.BlockSpec((_R, 64), lambda i: (i, 0)),
        out_shape=jax.ShapeDtypeStruct((N, 64), jnp.float32),
    )(x, W_enc, b_enc.reshape(1, 64), W1)


def _tc_scale1(degp_t, xw1):
    def body(dp_ref, xw_ref, dinv_ref, ta_ref, tb_ref):
        deg = dp_ref[...][:, 0:1] + dp_ref[...][:, 1:2] + 1.0
        dinv = lax.rsqrt(deg)
        dinv_ref[...] = dinv
        t = xw_ref[...] * dinv
        ta_ref[...] = t[:, :32]
        tb_ref[...] = t[:, 32:]
    return pl.pallas_call(
        body,
        grid=(_TG,),
        in_specs=[
            pl.BlockSpec((_R, 2), lambda i: (i, 0)),
            pl.BlockSpec((_R, 64), lambda i: (i, 0)),
        ],
        out_specs=[
            pl.BlockSpec((_R, 1), lambda i: (i, 0)),
            pl.BlockSpec((_R, 32), lambda i: (i, 0)),
            pl.BlockSpec((_R, 32), lambda i: (i, 0)),
        ],
        out_shape=[
            jax.ShapeDtypeStruct((N, 1), jnp.float32),
            jax.ShapeDtypeStruct((N, 32), jnp.float32),
            jax.ShapeDtypeStruct((N, 32), jnp.float32),
        ],
    )(degp_t, xw1)


def _tc_conv1_post(agg3, xw1, dinv, b1, W2):
    def body(aa_ref, ab_ref, xw_ref, dv_ref, b1_ref, w2_ref,
             xw2_ref, ta_ref, tb_ref):
        dv = dv_ref[...]
        agg = jnp.concatenate([aa_ref[0], ab_ref[0]], axis=1)
        h1 = dv * agg + (dv * dv) * xw_ref[...] + b1_ref[...]
        h1 = jnp.maximum(h1, 0.0)
        xw2 = jnp.dot(h1, w2_ref[...], preferred_element_type=jnp.float32)
        xw2_ref[...] = xw2
        t2 = xw2 * dv
        ta_ref[...] = t2[:, :16]
        tb_ref[...] = t2[:, 16:]
    return pl.pallas_call(
        body,
        grid=(_TG,),
        in_specs=[
            pl.BlockSpec((1, _R, 32), lambda i: (0, i, 0)),
            pl.BlockSpec((1, _R, 32), lambda i: (1, i, 0)),
            pl.BlockSpec((_R, 64), lambda i: (i, 0)),
            pl.BlockSpec((_R, 1), lambda i: (i, 0)),
            pl.BlockSpec((1, 64), lambda i: (0, 0)),
            pl.BlockSpec((64, 32), lambda i: (0, 0)),
        ],
        out_specs=[
            pl.BlockSpec((_R, 32), lambda i: (i, 0)),
            pl.BlockSpec((_R, 16), lambda i: (i, 0)),
            pl.BlockSpec((_R, 16), lambda i: (i, 0)),
        ],
        out_shape=[
            jax.ShapeDtypeStruct((N, 32), jnp.float32),
            jax.ShapeDtypeStruct((N, 16), jnp.float32),
            jax.ShapeDtypeStruct((N, 16), jnp.float32),
        ],
    )(agg3, agg3, xw1, dinv, b1.reshape(1, 64), W2)


def _tc_conv2_post(agg3, xw2, dinv, b2):
    def body(aa_ref, ab_ref, xw_ref, dv_ref, b2_ref, o_ref):
        dv = dv_ref[...]
        agg = jnp.concatenate([aa_ref[0], ab_ref[0]], axis=1)
        o_ref[...] = dv * agg + (dv * dv) * xw_ref[...] + b2_ref[...]
    return pl.pallas_call(
        body,
        grid=(_TG,),
        in_specs=[
            pl.BlockSpec((1, _R, 16), lambda i: (0, i, 0)),
            pl.BlockSpec((1, _R, 16), lambda i: (1, i, 0)),
            pl.BlockSpec((_R, 32), lambda i: (i, 0)),
            pl.BlockSpec((_R, 1), lambda i: (i, 0)),
            pl.BlockSpec((1, 32), lambda i: (0, 0)),
        ],
        out_specs=pl.BlockSpec((_R, 32), lambda i: (i, 0)),
        out_shape=jax.ShapeDtypeStruct((N, 32), jnp.float32),
    )(agg3, agg3, xw2, dinv, b2.reshape(1, 32))


def _tc_mlp(efs, efd, Wp1, bp1, Wp2, bp2):
    R2 = 8192
    def body(efs_ref, efd_ref, w1_ref, b1_ref, w2_ref, b2_ref, o_ref):
        ef = efs_ref[...] + efd_ref[...]
        e = jnp.dot(ef, w1_ref[...],
                    preferred_element_type=jnp.float32) + b1_ref[...]
        e = jnp.maximum(e, 0.0)
        z = jnp.dot(e, w2_ref[...],
                    preferred_element_type=jnp.float32) + b2_ref[...]
        o_ref[...] = 1.0 / (1.0 + jnp.exp(-z))
    return pl.pallas_call(
        body,
        grid=(ELP // R2,),
        in_specs=[
            pl.BlockSpec((R2, 32), lambda i: (i, 0)),
            pl.BlockSpec((R2, 32), lambda i: (i, 0)),
            pl.BlockSpec((32, 16), lambda i: (0, 0)),
            pl.BlockSpec((1, 16), lambda i: (0, 0)),
            pl.BlockSpec((16, 1), lambda i: (0, 0)),
            pl.BlockSpec((1, 1), lambda i: (0, 0)),
        ],
        out_specs=pl.BlockSpec((R2, 1), lambda i: (i, 0)),
        out_shape=jax.ShapeDtypeStruct((ELP, 1), jnp.float32),
    )(efs, efd, Wp1, bp1.reshape(1, 16), Wp2, bp2.reshape(1, 1))


def kernel(x, edge_index, edge_label_index,
           W_enc, b_enc, W1, b1, W2, b2, Wp1, bp1, Wp2, bp2):
    f32 = jnp.float32
    i32 = jnp.int32

    src = edge_index[0]
    dst = edge_index[1]
    src_p = jnp.concatenate(
        [src, jnp.zeros((EP - E,), i32)]).reshape(EP // CHUNK, CHUNK)
    dst_p = jnp.concatenate(
        [dst, jnp.full((EP - E,), N, i32)]).reshape(EP // CHUNK, CHUNK)
    lsrc_p = jnp.concatenate(
        [edge_label_index[0], jnp.zeros((ELP - EL,), i32)]
    ).reshape(ELP // CHUNK, CHUNK)
    ldst_p = jnp.concatenate(
        [edge_label_index[1], jnp.zeros((ELP - EL,), i32)]
    ).reshape(ELP // CHUNK, CHUNK)

    z1 = jnp.zeros((APT,), f32)
    z32 = jnp.zeros((ZC, 32), f32)
    z16 = jnp.zeros((ZC, 16), f32)
    ones = jnp.ones((CHUNK,), f32)

    xw1 = _tc_encoder(x, W_enc, b_enc, W1)
    degp = _deg_kernel(dst_p, ones, z1)
    degp_t = degp[:, :N].T
    dinv, tA, tB = _tc_scale1(degp_t, xw1)
    t3 = jnp.stack([tA, tB])
    agg3 = _conv_agg(t3, src_p, dst_p, z32, 32)
    xw2, t2A, t2B = _tc_conv1_post(agg3, xw1, dinv, b1, W2)
    t32 = jnp.stack([t2A, t2B])
    agg23 = _conv_agg(t32, src_p, dst_p, z16, 16)
    h2 = _tc_conv2_post(agg23, xw2, dinv, b2)
    efs, efd = _label_gather(h2, lsrc_p, ldst_p)
    out = _tc_mlp(efs, efd, Wp1, bp1, Wp2, bp2)
    return out[:EL, 0]


# speedup vs baseline: 1.8042x; 1.8042x over previous
import functools

import jax
import jax.numpy as jnp
from jax import lax
from jax.experimental import pallas as pl
from jax.experimental.pallas import tpu as pltpu
from jax.experimental.pallas import tpu_sc as plsc

N = 50000
E = 800000
EL = 200000
NC, NS = 2, 16
NW = NC * NS
CHUNK = 256
KE = 98
EP = NW * KE * CHUNK
IB = 7
KE0, KE1 = 98, 98
KL0, KL1 = 21, 28
ELP = NS * (KL0 + KL1) * CHUNK
LB = 7
ACC_ROWS = 50176
APT = ACC_ROWS // NS
ZC = 112


def _sc_mesh():
    return plsc.VectorSubcoreMesh(
        core_axis_name="c", subcore_axis_name="s",
        num_cores=NC, num_subcores=NS)


_SC_PARAMS = pltpu.CompilerParams(use_tc_tiling_on_sc=False)


def _deg_kernel(dst2d, ones, zrows):
    @functools.partial(
        pl.kernel,
        out_type=jax.ShapeDtypeStruct((NC, ACC_ROWS), jnp.float32),
        mesh=_sc_mesh(),
        compiler_params=_SC_PARAMS,
        scratch_types=[
            pltpu.VMEM((KE, CHUNK), jnp.int32),
            pltpu.VMEM((CHUNK,), jnp.float32),
            pltpu.VMEM((APT,), jnp.float32),
            pltpu.VMEM_SHARED((ACC_ROWS,), jnp.float32),
        ],
    )
    def deg(dst_hbm, ones_hbm, z_hbm, out_hbm, idx_v, ones_v, zbuf, acc):
        c = lax.axis_index("c")
        s = lax.axis_index("s")
        wid = c * NS + s
        sl = pl.ds(s * APT, APT)
        pltpu.sync_copy(z_hbm, zbuf)
        pltpu.sync_copy(zbuf, acc.at[sl])
        pltpu.sync_copy(ones_hbm, ones_v)
        pltpu.sync_copy(dst_hbm.at[pl.ds(wid * KE, KE)], idx_v)
        plsc.subcore_barrier()

        def body(j, carry):
            pltpu.sync_copy(ones_v, acc.at[idx_v.at[j]], add=True)
            return carry
        lax.fori_loop(0, KE, body, 0)
        plsc.subcore_barrier()
        pltpu.sync_copy(acc.at[sl], zbuf)
        pltpu.sync_copy(zbuf, out_hbm.at[c, sl])

    return deg(dst2d, ones, zrows)


def _conv_agg(t3, src2d, dst2d, zrows, F):
    @functools.partial(
        pl.kernel,
        out_type=jax.ShapeDtypeStruct((NC, ACC_ROWS, F), jnp.float32),
        mesh=_sc_mesh(),
        compiler_params=_SC_PARAMS,
        scratch_types=[
            pltpu.VMEM((IB, CHUNK), jnp.int32),
            pltpu.VMEM((IB, CHUNK), jnp.int32),
            pltpu.VMEM((CHUNK, F), jnp.float32),
            pltpu.VMEM((CHUNK, F), jnp.float32),
            pltpu.VMEM((ZC, F), jnp.float32),
            pltpu.SemaphoreType.DMA,
            pltpu.SemaphoreType.DMA,
            pltpu.SemaphoreType.DMA,
            pltpu.SemaphoreType.DMA,
            pltpu.VMEM_SHARED((ACC_ROWS, F), jnp.float32),
        ],
    )
    def agg(t3_hbm, src_hbm, dst_hbm, z_hbm, out_hbm,
            src_v, dst_v, rows0, rows1, zstage, sg0, sg1, ss0, ss1, acc):
        c = lax.axis_index("c")
        s = lax.axis_index("s")
        wid = c * NS + s
        tbl = t3_hbm.at[c]
        rows = (rows0, rows1)
        sg = (sg0, sg1)
        ss = (ss0, ss1)
        pltpu.sync_copy(z_hbm, zstage)

        def zbody(k, carry):
            pltpu.sync_copy(zstage, acc.at[pl.ds(s * APT + k * ZC, ZC)])
            return carry
        lax.fori_loop(0, APT // ZC, zbody, 0)
        plsc.subcore_barrier()

        base = jnp.where(c == 0, s * KE0, NS * KE0 + s * KE1)
        ng = jnp.where(c == 0, KE0 // IB, KE1 // IB)

        def outer(g, carry):
            gb = base + g * IB
            pltpu.sync_copy(src_hbm.at[pl.ds(gb, IB)], src_v)
            pltpu.sync_copy(dst_hbm.at[pl.ds(gb, IB)], dst_v)
            dg = [None, None]
            dsc = [None, None]
            dg[0] = pltpu.async_copy(tbl.at[src_v.at[0]], rows[0], sg[0])
            for k in range(IB):
                b = k % 2
                nb = 1 - b
                dg[b].wait()
                if k + 1 < IB:
                    if dsc[nb] is not None:
                        dsc[nb].wait()
                    dg[nb] = pltpu.async_copy(
                        tbl.at[src_v.at[k + 1]], rows[nb], sg[nb])
                dsc[b] = pltpu.async_copy(
                    rows[b], acc.at[dst_v.at[k]], ss[b], add=True)
            dsc[0].wait()
            dsc[1].wait()
            return carry
        lax.fori_loop(0, ng, outer, 0)
        plsc.subcore_barrier()

        def obody(k, carry):
            sk = pl.ds(s * APT + k * ZC, ZC)
            pltpu.sync_copy(acc.at[sk], zstage)
            pltpu.sync_copy(zstage, out_hbm.at[c, sk])
            return carry
        lax.fori_loop(0, APT // ZC, obody, 0)

    return agg(t3, src2d, dst2d, zrows)


def _label_gather(h2, lsrc2d, ldst2d):
    KLM = max(KL0, KL1)

    @functools.partial(
        pl.kernel,
        out_type=(jax.ShapeDtypeStruct((ELP, 32), jnp.bfloat16),
                  jax.ShapeDtypeStruct((ELP, 32), jnp.bfloat16)),
        mesh=_sc_mesh(),
        compiler_params=_SC_PARAMS,
        scratch_types=[
            pltpu.VMEM((KLM, CHUNK), jnp.int32),
            pltpu.VMEM((KLM, CHUNK), jnp.int32),
            pltpu.VMEM((CHUNK, 32), jnp.bfloat16),
            pltpu.VMEM((CHUNK, 32), jnp.bfloat16),
            pltpu.VMEM((CHUNK, 32), jnp.bfloat16),
            pltpu.VMEM((CHUNK, 32), jnp.bfloat16),
            pltpu.SemaphoreType.DMA,
            pltpu.SemaphoreType.DMA,
            pltpu.SemaphoreType.DMA,
            pltpu.SemaphoreType.DMA,
            pltpu.SemaphoreType.DMA,
            pltpu.SemaphoreType.DMA,
            pltpu.SemaphoreType.DMA,
            pltpu.SemaphoreType.DMA,
        ],
    )
    def lab(h2_hbm, src_hbm, dst_hbm, efs_hbm, efd_hbm, src_v, dst_v,
            bs0, bs1, bd0, bd1, gs0, gs1, gd0, gd1, ws0, ws1, wd0, wd1):
        c = lax.axis_index("c")
        s = lax.axis_index("s")
        base = jnp.where(c == 0, s * KL0, NS * KL0 + s * KL1)
        ng = jnp.where(c == 0, KL0 // LB, KL1 // LB)

        @pl.when(c == 0)
        def _():
            pltpu.sync_copy(src_hbm.at[pl.ds(s * KL0, KL0)],
                            src_v.at[pl.ds(0, KL0)])
            pltpu.sync_copy(dst_hbm.at[pl.ds(s * KL0, KL0)],
                            dst_v.at[pl.ds(0, KL0)])

        @pl.when(c == 1)
        def _():
            pltpu.sync_copy(src_hbm.at[pl.ds(NS * KL0 + s * KL1, KL1)],
                            src_v.at[pl.ds(0, KL1)])
            pltpu.sync_copy(dst_hbm.at[pl.ds(NS * KL0 + s * KL1, KL1)],
                            dst_v.at[pl.ds(0, KL1)])

        bufs_s = (bs0, bs1)
        bufs_d = (bd0, bd1)
        gs = (gs0, gs1)
        gd = (gd0, gd1)
        ws = (ws0, ws1)
        wd = (wd0, wd1)

        def outer(g, carry):
            jb = g * LB
            dgs = [None, None]
            dgd = [None, None]
            dss = [None, None]
            dsd = [None, None]
            dgs[0] = pltpu.async_copy(
                h2_hbm.at[src_v.at[jb]], bufs_s[0], gs[0])
            dgd[0] = pltpu.async_copy(
                h2_hbm.at[dst_v.at[jb]], bufs_d[0], gd[0])
            for k in range(LB):
                b = k % 2
                nb = 1 - b
                j = jb + k
                off = pl.ds((base + j) * CHUNK, CHUNK)
                dgs[b].wait()
                dgd[b].wait()
                if k + 1 < LB:
                    if dss[nb] is not None:
                        dss[nb].wait()
                    if dsd[nb] is not None:
                        dsd[nb].wait()
                    dgs[nb] = pltpu.async_copy(
                        h2_hbm.at[src_v.at[j + 1]], bufs_s[nb], gs[nb])
                    dgd[nb] = pltpu.async_copy(
                        h2_hbm.at[dst_v.at[j + 1]], bufs_d[nb], gd[nb])
                dss[b] = pltpu.async_copy(bufs_s[b], efs_hbm.at[off], ws[b])
                dsd[b] = pltpu.async_copy(bufs_d[b], efd_hbm.at[off], wd[b])
            dss[0].wait()
            dss[1].wait()
            dsd[0].wait()
            dsd[1].wait()
            return carry
        lax.fori_loop(0, ng, outer, 0)

    return lab(h2, lsrc2d, ldst2d)


_R = 1024
_TG = ACC_ROWS // _R


def _tc_encoder(x, W_enc, b_enc, W1):
    def body(x_ref, we_ref, be_ref, w1_ref, o_ref):
        h = jnp.dot(x_ref[...], we_ref[...],
                    preferred_element_type=jnp.float32) + be_ref[...]
        h = jnp.maximum(h, 0.0)
        o_ref[...] = jnp.dot(h, w1_ref[...],
                             preferred_element_type=jnp.float32)
    return pl.pallas_call(
        body,
        grid=(_TG,),
        in_specs=[
            pl.BlockSpec((_R, 128), lambda i: (i, 0)),
            pl.BlockSpec((128, 64), lambda i: (0, 0)),
            pl.BlockSpec((1, 64), lambda i: (0, 0)),
            pl.BlockSpec((64, 64), lambda i: (0, 0)),
        ],
        out_specs=pl.BlockSpec((_R, 64), lambda i: (i, 0)),
        out_shape=jax.ShapeDtypeStruct((N, 64), jnp.float32),
    )(x, W_enc, b_enc.reshape(1, 64), W1)


def _tc_scale1(degp_t, xw1):
    def body(dp_ref, xw_ref, dinv_ref, t3_ref):
        deg = dp_ref[...][:, 0:1] + dp_ref[...][:, 1:2] + 1.0
        dinv = lax.rsqrt(deg)
        dinv_ref[...] = dinv
        t = xw_ref[...] * dinv
        t3_ref[...] = jnp.stack([t[:, :32], t[:, 32:]], axis=0)
    return pl.pallas_call(
        body,
        grid=(_TG,),
        in_specs=[
            pl.BlockSpec((_R, 2), lambda i: (i, 0)),
            pl.BlockSpec((_R, 64), lambda i: (i, 0)),
        ],
        out_specs=[
            pl.BlockSpec((_R, 1), lambda i: (i, 0)),
            pl.BlockSpec((2, _R, 32), lambda i: (0, i, 0)),
        ],
        out_shape=[
            jax.ShapeDtypeStruct((N, 1), jnp.float32),
            jax.ShapeDtypeStruct((2, N, 32), jnp.float32),
        ],
    )(degp_t, xw1)


def _tc_conv1_post(agg3, xw1, dinv, b1, W2):
    def body(aa_ref, ab_ref, xw_ref, dv_ref, b1_ref, w2_ref,
             xw2_ref, t3_ref):
        dv = dv_ref[...]
        agg = jnp.concatenate([aa_ref[0], ab_ref[0]], axis=1)
        h1 = dv * agg + (dv * dv) * xw_ref[...] + b1_ref[...]
        h1 = jnp.maximum(h1, 0.0)
        xw2 = jnp.dot(h1, w2_ref[...], preferred_element_type=jnp.float32)
        xw2_ref[...] = xw2
        t2 = xw2 * dv
        t3_ref[...] = jnp.stack([t2[:, :16], t2[:, 16:]], axis=0)
    return pl.pallas_call(
        body,
        grid=(_TG,),
        in_specs=[
            pl.BlockSpec((1, _R, 32), lambda i: (0, i, 0)),
            pl.BlockSpec((1, _R, 32), lambda i: (1, i, 0)),
            pl.BlockSpec((_R, 64), lambda i: (i, 0)),
            pl.BlockSpec((_R, 1), lambda i: (i, 0)),
            pl.BlockSpec((1, 64), lambda i: (0, 0)),
            pl.BlockSpec((64, 32), lambda i: (0, 0)),
        ],
        out_specs=[
            pl.BlockSpec((_R, 32), lambda i: (i, 0)),
            pl.BlockSpec((2, _R, 16), lambda i: (0, i, 0)),
        ],
        out_shape=[
            jax.ShapeDtypeStruct((N, 32), jnp.float32),
            jax.ShapeDtypeStruct((2, N, 16), jnp.float32),
        ],
    )(agg3, agg3, xw1, dinv, b1.reshape(1, 64), W2)


def _tc_conv2_post(agg3, xw2, dinv, b2):
    def body(aa_ref, ab_ref, xw_ref, dv_ref, b2_ref, o_ref):
        dv = dv_ref[...]
        agg = jnp.concatenate([aa_ref[0], ab_ref[0]], axis=1)
        h2 = dv * agg + (dv * dv) * xw_ref[...] + b2_ref[...]
        o_ref[...] = h2.astype(jnp.bfloat16)
    return pl.pallas_call(
        body,
        grid=(_TG,),
        in_specs=[
            pl.BlockSpec((1, _R, 16), lambda i: (0, i, 0)),
            pl.BlockSpec((1, _R, 16), lambda i: (1, i, 0)),
            pl.BlockSpec((_R, 32), lambda i: (i, 0)),
            pl.BlockSpec((_R, 1), lambda i: (i, 0)),
            pl.BlockSpec((1, 32), lambda i: (0, 0)),
        ],
        out_specs=pl.BlockSpec((_R, 32), lambda i: (i, 0)),
        out_shape=jax.ShapeDtypeStruct((N, 32), jnp.bfloat16),
    )(agg3, agg3, xw2, dinv, b2.reshape(1, 32))


def _tc_mlp(efs, efd, Wp1, bp1, Wp2, bp2):
    R2 = 7168
    def body(efs_ref, efd_ref, w1_ref, b1_ref, w2_ref, b2_ref, o_ref):
        ef = (efs_ref[...].astype(jnp.float32)
              + efd_ref[...].astype(jnp.float32))
        e = jnp.dot(ef, w1_ref[...],
                    preferred_element_type=jnp.float32) + b1_ref[...]
        e = jnp.maximum(e, 0.0)
        z = jnp.dot(e, w2_ref[...],
                    preferred_element_type=jnp.float32) + b2_ref[...]
        o_ref[...] = 1.0 / (1.0 + jnp.exp(-z))
    return pl.pallas_call(
        body,
        grid=(ELP // R2,),
        in_specs=[
            pl.BlockSpec((R2, 32), lambda i: (i, 0)),
            pl.BlockSpec((R2, 32), lambda i: (i, 0)),
            pl.BlockSpec((32, 16), lambda i: (0, 0)),
            pl.BlockSpec((1, 16), lambda i: (0, 0)),
            pl.BlockSpec((16, 1), lambda i: (0, 0)),
            pl.BlockSpec((1, 1), lambda i: (0, 0)),
        ],
        out_specs=pl.BlockSpec((R2, 1), lambda i: (i, 0)),
        out_shape=jax.ShapeDtypeStruct((ELP, 1), jnp.float32),
    )(efs, efd, Wp1, bp1.reshape(1, 16), Wp2, bp2.reshape(1, 1))


def kernel(x, edge_index, edge_label_index,
           W_enc, b_enc, W1, b1, W2, b2, Wp1, bp1, Wp2, bp2):
    f32 = jnp.float32
    i32 = jnp.int32

    src = edge_index[0]
    dst = edge_index[1]
    src_p = jnp.concatenate(
        [src, jnp.zeros((EP - E,), i32)]).reshape(EP // CHUNK, CHUNK)
    dst_p = jnp.concatenate(
        [dst, jnp.full((EP - E,), N, i32)]).reshape(EP // CHUNK, CHUNK)
    lsrc_p = jnp.concatenate(
        [edge_label_index[0], jnp.zeros((ELP - EL,), i32)]
    ).reshape(ELP // CHUNK, CHUNK)
    ldst_p = jnp.concatenate(
        [edge_label_index[1], jnp.zeros((ELP - EL,), i32)]
    ).reshape(ELP // CHUNK, CHUNK)

    z1 = jnp.zeros((APT,), f32)
    z32 = jnp.zeros((ZC, 32), f32)
    z16 = jnp.zeros((ZC, 16), f32)
    ones = jnp.ones((CHUNK,), f32)

    xw1 = _tc_encoder(x, W_enc, b_enc, W1)
    degp = _deg_kernel(dst_p, ones, z1)
    degp_t = degp[:, :N].T
    dinv, t3 = _tc_scale1(degp_t, xw1)
    agg3 = _conv_agg(t3, src_p, dst_p, z32, 32)
    xw2, t32 = _tc_conv1_post(agg3, xw1, dinv, b1, W2)
    agg23 = _conv_agg(t32, src_p, dst_p, z16, 16)
    h2 = _tc_conv2_post(agg23, xw2, dinv, b2)
    efs, efd = _label_gather(h2, lsrc_p, ldst_p)
    out = _tc_mlp(efs, efd, Wp1, bp1, Wp2, bp2)
    return out[:EL, 0]


# speedup vs baseline: 2.0106x; 1.1144x over previous
import functools

import jax
import jax.numpy as jnp
from jax import lax
from jax.experimental import pallas as pl
from jax.experimental.pallas import tpu as pltpu
from jax.experimental.pallas import tpu_sc as plsc

N = 50000
E = 800000
EL = 200000
NC, NS = 2, 16
NW = NC * NS
CHUNK = 256
KE = 98
EP = NW * KE * CHUNK
IB = 7
KE0, KE1 = 98, 98
KL0, KL1 = 21, 28
ELP = NS * (KL0 + KL1) * CHUNK
LB = 7
ACC_ROWS = 50176
APT = ACC_ROWS // NS
ZC = 112


def _sc_mesh():
    return plsc.VectorSubcoreMesh(
        core_axis_name="c", subcore_axis_name="s",
        num_cores=NC, num_subcores=NS)


_SC_PARAMS = pltpu.CompilerParams(use_tc_tiling_on_sc=False)


def _deg_kernel(dst2d, ones, zrows):
    @functools.partial(
        pl.kernel,
        out_type=jax.ShapeDtypeStruct((NC, ACC_ROWS), jnp.float32),
        mesh=_sc_mesh(),
        compiler_params=_SC_PARAMS,
        scratch_types=[
            pltpu.VMEM((KE, CHUNK), jnp.int32),
            pltpu.VMEM((CHUNK,), jnp.float32),
            pltpu.VMEM((APT,), jnp.float32),
            pltpu.VMEM_SHARED((ACC_ROWS,), jnp.float32),
        ],
    )
    def deg(dst_hbm, ones_hbm, z_hbm, out_hbm, idx_v, ones_v, zbuf, acc):
        c = lax.axis_index("c")
        s = lax.axis_index("s")
        wid = c * NS + s
        sl = pl.ds(s * APT, APT)
        pltpu.sync_copy(z_hbm, zbuf)
        pltpu.sync_copy(zbuf, acc.at[sl])
        pltpu.sync_copy(ones_hbm, ones_v)
        pltpu.sync_copy(dst_hbm.at[pl.ds(wid * KE, KE)], idx_v)
        plsc.subcore_barrier()

        def body(j, carry):
            pltpu.sync_copy(ones_v, acc.at[idx_v.at[j]], add=True)
            return carry
        lax.fori_loop(0, KE, body, 0)
        plsc.subcore_barrier()
        pltpu.sync_copy(acc.at[sl], zbuf)
        pltpu.sync_copy(zbuf, out_hbm.at[c, sl])

    return deg(dst2d, ones, zrows)


def _conv_agg(t3, src2d, dst2d, zrows, F):
    @functools.partial(
        pl.kernel,
        out_type=jax.ShapeDtypeStruct((NC, ACC_ROWS, F), jnp.float32),
        mesh=_sc_mesh(),
        compiler_params=_SC_PARAMS,
        scratch_types=[
            pltpu.VMEM((IB, CHUNK), jnp.int32),
            pltpu.VMEM((IB, CHUNK), jnp.int32),
            pltpu.VMEM((CHUNK, F), jnp.float32),
            pltpu.VMEM((CHUNK, F), jnp.float32),
            pltpu.VMEM((ZC, F), jnp.float32),
            pltpu.SemaphoreType.DMA,
            pltpu.SemaphoreType.DMA,
            pltpu.SemaphoreType.DMA,
            pltpu.SemaphoreType.DMA,
            pltpu.VMEM_SHARED((ACC_ROWS, F), jnp.float32),
        ],
    )
    def agg(t3_hbm, src_hbm, dst_hbm, z_hbm, out_hbm,
            src_v, dst_v, rows0, rows1, zstage, sg0, sg1, ss0, ss1, acc):
        c = lax.axis_index("c")
        s = lax.axis_index("s")
        wid = c * NS + s
        tbl = t3_hbm.at[c]
        rows = (rows0, rows1)
        sg = (sg0, sg1)
        ss = (ss0, ss1)
        pltpu.sync_copy(z_hbm, zstage)

        def zbody(k, carry):
            pltpu.sync_copy(zstage, acc.at[pl.ds(s * APT + k * ZC, ZC)])
            return carry
        lax.fori_loop(0, APT // ZC, zbody, 0)
        plsc.subcore_barrier()

        base = jnp.where(c == 0, s * KE0, NS * KE0 + s * KE1)
        ng = jnp.where(c == 0, KE0 // IB, KE1 // IB)

        def outer(g, carry):
            gb = base + g * IB
            pltpu.sync_copy(src_hbm.at[pl.ds(gb, IB)], src_v)
            pltpu.sync_copy(dst_hbm.at[pl.ds(gb, IB)], dst_v)
            dg = [None, None]
            dsc = [None, None]
            dg[0] = pltpu.async_copy(tbl.at[src_v.at[0]], rows[0], sg[0])
            for k in range(IB):
                b = k % 2
                nb = 1 - b
                dg[b].wait()
                if k + 1 < IB:
                    if dsc[nb] is not None:
                        dsc[nb].wait()
                    dg[nb] = pltpu.async_copy(
                        tbl.at[src_v.at[k + 1]], rows[nb], sg[nb])
                dsc[b] = pltpu.async_copy(
                    rows[b], acc.at[dst_v.at[k]], ss[b], add=True)
            dsc[0].wait()
            dsc[1].wait()
            return carry
        lax.fori_loop(0, ng, outer, 0)
        plsc.subcore_barrier()

        def obody(k, carry):
            sk = pl.ds(s * APT + k * ZC, ZC)
            pltpu.sync_copy(acc.at[sk], zstage)
            pltpu.sync_copy(zstage, out_hbm.at[c, sk])
            return carry
        lax.fori_loop(0, APT // ZC, obody, 0)

    return agg(t3, src2d, dst2d, zrows)


def _label_gather(h2, lsrc2d, ldst2d):
    KLM = max(KL0, KL1)

    @functools.partial(
        pl.kernel,
        out_type=(jax.ShapeDtypeStruct((ELP, 32), jnp.bfloat16),
                  jax.ShapeDtypeStruct((ELP, 32), jnp.bfloat16)),
        mesh=_sc_mesh(),
        compiler_params=_SC_PARAMS,
        scratch_types=[
            pltpu.VMEM((KLM, CHUNK), jnp.int32),
            pltpu.VMEM((KLM, CHUNK), jnp.int32),
            pltpu.VMEM((CHUNK, 32), jnp.bfloat16),
            pltpu.VMEM((CHUNK, 32), jnp.bfloat16),
            pltpu.VMEM((CHUNK, 32), jnp.bfloat16),
            pltpu.VMEM((CHUNK, 32), jnp.bfloat16),
            pltpu.SemaphoreType.DMA,
            pltpu.SemaphoreType.DMA,
            pltpu.SemaphoreType.DMA,
            pltpu.SemaphoreType.DMA,
            pltpu.SemaphoreType.DMA,
            pltpu.SemaphoreType.DMA,
            pltpu.SemaphoreType.DMA,
            pltpu.SemaphoreType.DMA,
        ],
    )
    def lab(h2_hbm, src_hbm, dst_hbm, efs_hbm, efd_hbm, src_v, dst_v,
            bs0, bs1, bd0, bd1, gs0, gs1, gd0, gd1, ws0, ws1, wd0, wd1):
        c = lax.axis_index("c")
        s = lax.axis_index("s")
        base = jnp.where(c == 0, s * KL0, NS * KL0 + s * KL1)
        ng = jnp.where(c == 0, KL0 // LB, KL1 // LB)

        @pl.when(c == 0)
        def _():
            pltpu.sync_copy(src_hbm.at[pl.ds(s * KL0, KL0)],
                            src_v.at[pl.ds(0, KL0)])
            pltpu.sync_copy(dst_hbm.at[pl.ds(s * KL0, KL0)],
                            dst_v.at[pl.ds(0, KL0)])

        @pl.when(c == 1)
        def _():
            pltpu.sync_copy(src_hbm.at[pl.ds(NS * KL0 + s * KL1, KL1)],
                            src_v.at[pl.ds(0, KL1)])
            pltpu.sync_copy(dst_hbm.at[pl.ds(NS * KL0 + s * KL1, KL1)],
                            dst_v.at[pl.ds(0, KL1)])

        bufs_s = (bs0, bs1)
        bufs_d = (bd0, bd1)
        gs = (gs0, gs1)
        gd = (gd0, gd1)
        ws = (ws0, ws1)
        wd = (wd0, wd1)

        def outer(g, carry):
            jb = g * LB
            dgs = [None, None]
            dgd = [None, None]
            dss = [None, None]
            dsd = [None, None]
            dgs[0] = pltpu.async_copy(
                h2_hbm.at[src_v.at[jb]], bufs_s[0], gs[0])
            dgd[0] = pltpu.async_copy(
                h2_hbm.at[dst_v.at[jb]], bufs_d[0], gd[0])
            for k in range(LB):
                b = k % 2
                nb = 1 - b
                j = jb + k
                off = pl.ds((base + j) * CHUNK, CHUNK)
                dgs[b].wait()
                dgd[b].wait()
                if k + 1 < LB:
                    if dss[nb] is not None:
                        dss[nb].wait()
                    if dsd[nb] is not None:
                        dsd[nb].wait()
                    dgs[nb] = pltpu.async_copy(
                        h2_hbm.at[src_v.at[j + 1]], bufs_s[nb], gs[nb])
                    dgd[nb] = pltpu.async_copy(
                        h2_hbm.at[dst_v.at[j + 1]], bufs_d[nb], gd[nb])
                dss[b] = pltpu.async_copy(bufs_s[b], efs_hbm.at[off], ws[b])
                dsd[b] = pltpu.async_copy(bufs_d[b], efd_hbm.at[off], wd[b])
            dss[0].wait()
            dss[1].wait()
            dsd[0].wait()
            dsd[1].wait()
            return carry
        lax.fori_loop(0, ng, outer, 0)

    return lab(h2, lsrc2d, ldst2d)


_R = 1024
_TG = ACC_ROWS // _R


def _tc_encoder(x, degp_t, W_enc, b_enc, W1):
    def body(x_ref, dp_ref, we_ref, be_ref, w1_ref,
             xw1_ref, dinv_ref, t3_ref):
        h = jnp.dot(x_ref[...], we_ref[...],
                    preferred_element_type=jnp.float32) + be_ref[...]
        h = jnp.maximum(h, 0.0)
        xw1 = jnp.dot(h, w1_ref[...], preferred_element_type=jnp.float32)
        xw1_ref[...] = xw1
        deg = dp_ref[...][:, 0:1] + dp_ref[...][:, 1:2] + 1.0
        dinv = lax.rsqrt(deg)
        dinv_ref[...] = dinv
        t = xw1 * dinv
        t3_ref[...] = jnp.stack([t[:, :32], t[:, 32:]], axis=0)
    return pl.pallas_call(
        body,
        grid=(_TG,),
        in_specs=[
            pl.BlockSpec((_R, 128), lambda i: (i, 0)),
            pl.BlockSpec((_R, 2), lambda i: (i, 0)),
            pl.BlockSpec((128, 64), lambda i: (0, 0)),
            pl.BlockSpec((1, 64), lambda i: (0, 0)),
            pl.BlockSpec((64, 64), lambda i: (0, 0)),
        ],
        out_specs=[
            pl.BlockSpec((_R, 64), lambda i: (i, 0)),
            pl.BlockSpec((_R, 1), lambda i: (i, 0)),
            pl.BlockSpec((2, _R, 32), lambda i: (0, i, 0)),
        ],
        out_shape=[
            jax.ShapeDtypeStruct((N, 64), jnp.float32),
            jax.ShapeDtypeStruct((N, 1), jnp.float32),
            jax.ShapeDtypeStruct((2, N, 32), jnp.float32),
        ],
    )(x, degp_t, W_enc, b_enc.reshape(1, 64), W1)


def _tc_conv1_post(agg3, xw1, dinv, b1, W2):
    def body(aa_ref, ab_ref, xw_ref, dv_ref, b1_ref, w2_ref,
             xw2_ref, t3_ref):
        dv = dv_ref[...]
        agg = jnp.concatenate([aa_ref[0], ab_ref[0]], axis=1)
        h1 = dv * agg + (dv * dv) * xw_ref[...] + b1_ref[...]
        h1 = jnp.maximum(h1, 0.0)
        xw2 = jnp.dot(h1, w2_ref[...], preferred_element_type=jnp.float32)
        xw2_ref[...] = xw2
        t2 = xw2 * dv
        t3_ref[...] = jnp.stack([t2[:, :16], t2[:, 16:]], axis=0)
    return pl.pallas_call(
        body,
        grid=(_TG,),
        in_specs=[
            pl.BlockSpec((1, _R, 32), lambda i: (0, i, 0)),
            pl.BlockSpec((1, _R, 32), lambda i: (1, i, 0)),
            pl.BlockSpec((_R, 64), lambda i: (i, 0)),
            pl.BlockSpec((_R, 1), lambda i: (i, 0)),
            pl.BlockSpec((1, 64), lambda i: (0, 0)),
            pl.BlockSpec((64, 32), lambda i: (0, 0)),
        ],
        out_specs=[
            pl.BlockSpec((_R, 32), lambda i: (i, 0)),
            pl.BlockSpec((2, _R, 16), lambda i: (0, i, 0)),
        ],
        out_shape=[
            jax.ShapeDtypeStruct((N, 32), jnp.float32),
            jax.ShapeDtypeStruct((2, N, 16), jnp.float32),
        ],
    )(agg3, agg3, xw1, dinv, b1.reshape(1, 64), W2)


def _tc_conv2_post(agg3, xw2, dinv, b2):
    def body(aa_ref, ab_ref, xw_ref, dv_ref, b2_ref, o_ref):
        dv = dv_ref[...]
        agg = jnp.concatenate([aa_ref[0], ab_ref[0]], axis=1)
        h2 = dv * agg + (dv * dv) * xw_ref[...] + b2_ref[...]
        o_ref[...] = h2.astype(jnp.bfloat16)
    return pl.pallas_call(
        body,
        grid=(_TG,),
        in_specs=[
            pl.BlockSpec((1, _R, 16), lambda i: (0, i, 0)),
            pl.BlockSpec((1, _R, 16), lambda i: (1, i, 0)),
            pl.BlockSpec((_R, 32), lambda i: (i, 0)),
            pl.BlockSpec((_R, 1), lambda i: (i, 0)),
            pl.BlockSpec((1, 32), lambda i: (0, 0)),
        ],
        out_specs=pl.BlockSpec((_R, 32), lambda i: (i, 0)),
        out_shape=jax.ShapeDtypeStruct((N, 32), jnp.bfloat16),
    )(agg3, agg3, xw2, dinv, b2.reshape(1, 32))


def _tc_mlp(efsp, efdp, Wp1, bp1, Wp2, bp2):
    EP4 = ELP // 4
    R2 = 1792
    eye4 = jnp.eye(4, dtype=jnp.float32)
    W1b = jnp.kron(eye4, Wp1)
    b1b = jnp.tile(bp1, 4).reshape(1, 64)
    W2b = jnp.kron(eye4, Wp2)
    b2b = jnp.tile(bp2, 4).reshape(1, 4)

    def body(efs_ref, efd_ref, w1_ref, b1_ref, w2_ref, b2_ref, o_ref):
        ef = (efs_ref[...].astype(jnp.float32)
              + efd_ref[...].astype(jnp.float32))
        e = jnp.dot(ef, w1_ref[...],
                    preferred_element_type=jnp.float32) + b1_ref[...]
        e = jnp.maximum(e, 0.0)
        z = jnp.dot(e, w2_ref[...],
                    preferred_element_type=jnp.float32) + b2_ref[...]
        o_ref[...] = 1.0 / (1.0 + jnp.exp(-z))
    return pl.pallas_call(
        body,
        grid=(EP4 // R2,),
        in_specs=[
            pl.BlockSpec((R2, 128), lambda i: (i, 0)),
            pl.BlockSpec((R2, 128), lambda i: (i, 0)),
            pl.BlockSpec((128, 64), lambda i: (0, 0)),
            pl.BlockSpec((1, 64), lambda i: (0, 0)),
            pl.BlockSpec((64, 4), lambda i: (0, 0)),
            pl.BlockSpec((1, 4), lambda i: (0, 0)),
        ],
        out_specs=pl.BlockSpec((R2, 4), lambda i: (i, 0)),
        out_shape=jax.ShapeDtypeStruct((EP4, 4), jnp.float32),
    )(efsp, efdp, W1b, b1b, W2b, b2b)


def kernel(x, edge_index, edge_label_index,
           W_enc, b_enc, W1, b1, W2, b2, Wp1, bp1, Wp2, bp2):
    f32 = jnp.float32
    i32 = jnp.int32

    src = edge_index[0]
    dst = edge_index[1]
    src_p = jnp.concatenate(
        [src, jnp.zeros((EP - E,), i32)]).reshape(EP // CHUNK, CHUNK)
    dst_p = jnp.concatenate(
        [dst, jnp.full((EP - E,), N, i32)]).reshape(EP // CHUNK, CHUNK)
    lsrc_p = jnp.concatenate(
        [edge_label_index[0], jnp.zeros((ELP - EL,), i32)]
    ).reshape(ELP // CHUNK, CHUNK)
    ldst_p = jnp.concatenate(
        [edge_label_index[1], jnp.zeros((ELP - EL,), i32)]
    ).reshape(ELP // CHUNK, CHUNK)

    z1 = jnp.zeros((APT,), f32)
    z32 = jnp.zeros((ZC, 32), f32)
    z16 = jnp.zeros((ZC, 16), f32)
    ones = jnp.ones((CHUNK,), f32)

    degp = _deg_kernel(dst_p, ones, z1)
    degp_t = degp[:, :N].T
    xw1, dinv, t3 = _tc_encoder(x, degp_t, W_enc, b_enc, W1)
    agg3 = _conv_agg(t3, src_p, dst_p, z32, 32)
    xw2, t32 = _tc_conv1_post(agg3, xw1, dinv, b1, W2)
    agg23 = _conv_agg(t32, src_p, dst_p, z16, 16)
    h2 = _tc_conv2_post(agg23, xw2, dinv, b2)
    efs, efd = _label_gather(h2, lsrc_p, ldst_p)
    efsp = efs.reshape(ELP // 4, 128)
    efdp = efd.reshape(ELP // 4, 128)
    out = _tc_mlp(efsp, efdp, Wp1, bp1, Wp2, bp2)
    return out.reshape(ELP)[:EL]


# speedup vs baseline: 2.1854x; 1.0869x over previous
import functools

import jax
import jax.numpy as jnp
from jax import lax
from jax.experimental import pallas as pl
from jax.experimental.pallas import tpu as pltpu
from jax.experimental.pallas import tpu_sc as plsc

N = 50000
E = 800000
EL = 200000
NC, NS = 2, 16
NW = NC * NS
CHUNK = 256
KE = 98
EP = NW * KE * CHUNK
IB = 7
KE0, KE1 = 98, 98
KL0, KL1 = 21, 28
ELP = NS * (KL0 + KL1) * CHUNK
LB = 7
ACC_ROWS = 50176
APT = ACC_ROWS // NS
ZC = 112


def _sc_mesh():
    return plsc.VectorSubcoreMesh(
        core_axis_name="c", subcore_axis_name="s",
        num_cores=NC, num_subcores=NS)


_SC_PARAMS = pltpu.CompilerParams(use_tc_tiling_on_sc=False)


def _deg_kernel(dst2d, ones, zrows):
    @functools.partial(
        pl.kernel,
        out_type=jax.ShapeDtypeStruct((NC, ACC_ROWS), jnp.float32),
        mesh=_sc_mesh(),
        compiler_params=_SC_PARAMS,
        scratch_types=[
            pltpu.VMEM((KE, CHUNK), jnp.int32),
            pltpu.VMEM((CHUNK,), jnp.float32),
            pltpu.VMEM((APT,), jnp.float32),
            pltpu.VMEM_SHARED((ACC_ROWS,), jnp.float32),
        ],
    )
    def deg(dst_hbm, ones_hbm, z_hbm, out_hbm, idx_v, ones_v, zbuf, acc):
        c = lax.axis_index("c")
        s = lax.axis_index("s")
        wid = c * NS + s
        sl = pl.ds(s * APT, APT)
        pltpu.sync_copy(z_hbm, zbuf)
        pltpu.sync_copy(zbuf, acc.at[sl])
        pltpu.sync_copy(ones_hbm, ones_v)
        pltpu.sync_copy(dst_hbm.at[pl.ds(wid * KE, KE)], idx_v)
        plsc.subcore_barrier()

        def body(j, carry):
            pltpu.sync_copy(ones_v, acc.at[idx_v.at[j]], add=True)
            return carry
        lax.fori_loop(0, KE, body, 0)
        plsc.subcore_barrier()
        pltpu.sync_copy(acc.at[sl], zbuf)
        pltpu.sync_copy(zbuf, out_hbm.at[c, sl])

    return deg(dst2d, ones, zrows)


def _conv_agg(t3, src2d, dst2d, zrows, F):
    @functools.partial(
        pl.kernel,
        out_type=jax.ShapeDtypeStruct((NC, ACC_ROWS, F), jnp.float32),
        mesh=_sc_mesh(),
        compiler_params=_SC_PARAMS,
        scratch_types=[
            pltpu.VMEM((IB, CHUNK), jnp.int32),
            pltpu.VMEM((IB, CHUNK), jnp.int32),
            pltpu.VMEM((CHUNK, F), jnp.float32),
            pltpu.VMEM((CHUNK, F), jnp.float32),
            pltpu.VMEM((ZC, F), jnp.float32),
            pltpu.SemaphoreType.DMA,
            pltpu.SemaphoreType.DMA,
            pltpu.SemaphoreType.DMA,
            pltpu.SemaphoreType.DMA,
            pltpu.VMEM_SHARED((ACC_ROWS, F), jnp.float32),
        ],
    )
    def agg(t3_hbm, src_hbm, dst_hbm, z_hbm, out_hbm,
            src_v, dst_v, rows0, rows1, zstage, sg0, sg1, ss0, ss1, acc):
        c = lax.axis_index("c")
        s = lax.axis_index("s")
        wid = c * NS + s
        tbl = t3_hbm.at[c]
        rows = (rows0, rows1)
        sg = (sg0, sg1)
        ss = (ss0, ss1)
        pltpu.sync_copy(z_hbm, zstage)

        def zbody(k, carry):
            pltpu.sync_copy(zstage, acc.at[pl.ds(s * APT + k * ZC, ZC)])
            return carry
        lax.fori_loop(0, APT // ZC, zbody, 0)
        plsc.subcore_barrier()

        base = jnp.where(c == 0, s * KE0, NS * KE0 + s * KE1)
        ng = jnp.where(c == 0, KE0 // IB, KE1 // IB)

        def outer(g, carry):
            gb = base + g * IB
            pltpu.sync_copy(src_hbm.at[pl.ds(gb, IB)], src_v)
            pltpu.sync_copy(dst_hbm.at[pl.ds(gb, IB)], dst_v)
            dg = [None, None]
            dsc = [None, None]
            dg[0] = pltpu.async_copy(tbl.at[src_v.at[0]], rows[0], sg[0])
            for k in range(IB):
                b = k % 2
                nb = 1 - b
                dg[b].wait()
                if k + 1 < IB:
                    if dsc[nb] is not None:
                        dsc[nb].wait()
                    dg[nb] = pltpu.async_copy(
                        tbl.at[src_v.at[k + 1]], rows[nb], sg[nb])
                dsc[b] = pltpu.async_copy(
                    rows[b], acc.at[dst_v.at[k]], ss[b], add=True)
            dsc[0].wait()
            dsc[1].wait()
            return carry
        lax.fori_loop(0, ng, outer, 0)
        plsc.subcore_barrier()

        def obody(k, carry):
            sk = pl.ds(s * APT + k * ZC, ZC)
            pltpu.sync_copy(acc.at[sk], zstage)
            pltpu.sync_copy(zstage, out_hbm.at[c, sk])
            return carry
        lax.fori_loop(0, APT // ZC, obody, 0)

    return agg(t3, src2d, dst2d, zrows)


def _label_gather(h2, lsrc2d, ldst2d):
    KLM = max(KL0, KL1)
    C4 = CHUNK // 4

    @functools.partial(
        pl.kernel,
        out_type=jax.ShapeDtypeStruct((ELP // 4, 128), jnp.bfloat16),
        mesh=_sc_mesh(),
        compiler_params=_SC_PARAMS,
        scratch_types=[
            pltpu.VMEM((KLM, CHUNK), jnp.int32),
            pltpu.VMEM((KLM, CHUNK), jnp.int32),
            pltpu.VMEM((CHUNK, 32), jnp.bfloat16),
            pltpu.VMEM((CHUNK, 32), jnp.bfloat16),
            pltpu.VMEM((CHUNK, 32), jnp.bfloat16),
            pltpu.VMEM((CHUNK, 32), jnp.bfloat16),
            pltpu.VMEM((C4, 128), jnp.bfloat16),
            pltpu.VMEM((C4, 128), jnp.bfloat16),
            pltpu.SemaphoreType.DMA,
            pltpu.SemaphoreType.DMA,
            pltpu.SemaphoreType.DMA,
            pltpu.SemaphoreType.DMA,
            pltpu.SemaphoreType.DMA,
            pltpu.SemaphoreType.DMA,
        ],
    )
    def lab(h2_hbm, src_hbm, dst_hbm, ef_hbm, src_v, dst_v,
            bs0, bs1, bd0, bd1, bp0, bp1, gs0, gs1, gd0, gd1, ws0, ws1):
        c = lax.axis_index("c")
        s = lax.axis_index("s")
        base = jnp.where(c == 0, s * KL0, NS * KL0 + s * KL1)
        ng = jnp.where(c == 0, KL0 // LB, KL1 // LB)

        @pl.when(c == 0)
        def _():
            pltpu.sync_copy(src_hbm.at[pl.ds(s * KL0, KL0)],
                            src_v.at[pl.ds(0, KL0)])
            pltpu.sync_copy(dst_hbm.at[pl.ds(s * KL0, KL0)],
                            dst_v.at[pl.ds(0, KL0)])

        @pl.when(c == 1)
        def _():
            pltpu.sync_copy(src_hbm.at[pl.ds(NS * KL0 + s * KL1, KL1)],
                            src_v.at[pl.ds(0, KL1)])
            pltpu.sync_copy(dst_hbm.at[pl.ds(NS * KL0 + s * KL1, KL1)],
                            dst_v.at[pl.ds(0, KL1)])

        bufs_s = (bs0, bs1)
        bufs_d = (bd0, bd1)
        bufs_p = (bp0, bp1)
        gs = (gs0, gs1)
        gd = (gd0, gd1)
        ws = (ws0, ws1)

        def outer(g, carry):
            jb = g * LB
            dgs = [None, None]
            dgd = [None, None]
            dsp = [None, None]
            dgs[0] = pltpu.async_copy(
                h2_hbm.at[src_v.at[jb]], bufs_s[0], gs[0])
            dgd[0] = pltpu.async_copy(
                h2_hbm.at[dst_v.at[jb]], bufs_d[0], gd[0])
            for k in range(LB):
                b = k % 2
                nb = 1 - b
                j = jb + k
                dgs[b].wait()
                dgd[b].wait()
                if k + 1 < LB:
                    if dsp[nb] is not None:
                        dsp[nb].wait()
                    dgs[nb] = pltpu.async_copy(
                        h2_hbm.at[src_v.at[j + 1]], bufs_s[nb], gs[nb])
                    dgd[nb] = pltpu.async_copy(
                        h2_hbm.at[dst_v.at[j + 1]], bufs_d[nb], gd[nb])
                bsb, bdb, bpb = bufs_s[b], bufs_d[b], bufs_p[b]

                def repack(q, carry2):
                    for sub in range(4):
                        v = bsb[4 * q + sub, :] + bdb[4 * q + sub, :]
                        bpb[q, pl.ds(sub * 32, 32)] = v
                    return carry2
                lax.fori_loop(0, C4, repack, 0)
                dsp[b] = pltpu.async_copy(
                    bpb, ef_hbm.at[pl.ds((base + j) * C4, C4)], ws[b])
            dsp[0].wait()
            dsp[1].wait()
            return carry
        lax.fori_loop(0, ng, outer, 0)

    return lab(h2, lsrc2d, ldst2d)


_R = 1024
_TG = ACC_ROWS // _R


def _tc_encoder(x, degp_t, W_enc, b_enc, W1):
    def body(x_ref, dp_ref, we_ref, be_ref, w1_ref,
             xw1_ref, dinv_ref, t3_ref):
        h = jnp.dot(x_ref[...], we_ref[...],
                    preferred_element_type=jnp.float32) + be_ref[...]
        h = jnp.maximum(h, 0.0)
        xw1 = jnp.dot(h, w1_ref[...], preferred_element_type=jnp.float32)
        xw1_ref[...] = xw1
        deg = dp_ref[...][:, 0:1] + dp_ref[...][:, 1:2] + 1.0
        dinv = lax.rsqrt(deg)
        dinv_ref[...] = dinv
        t = xw1 * dinv
        t3_ref[...] = jnp.stack([t[:, :32], t[:, 32:]], axis=0)
    return pl.pallas_call(
        body,
        grid=(_TG,),
        in_specs=[
            pl.BlockSpec((_R, 128), lambda i: (i, 0)),
            pl.BlockSpec((_R, 2), lambda i: (i, 0)),
            pl.BlockSpec((128, 64), lambda i: (0, 0)),
            pl.BlockSpec((1, 64), lambda i: (0, 0)),
            pl.BlockSpec((64, 64), lambda i: (0, 0)),
        ],
        out_specs=[
            pl.BlockSpec((_R, 64), lambda i: (i, 0)),
            pl.BlockSpec((_R, 1), lambda i: (i, 0)),
            pl.BlockSpec((2, _R, 32), lambda i: (0, i, 0)),
        ],
        out_shape=[
            jax.ShapeDtypeStruct((N, 64), jnp.float32),
            jax.ShapeDtypeStruct((N, 1), jnp.float32),
            jax.ShapeDtypeStruct((2, N, 32), jnp.float32),
        ],
    )(x, degp_t, W_enc, b_enc.reshape(1, 64), W1)


def _tc_conv1_post(agg3, xw1, dinv, b1, W2):
    def body(aa_ref, ab_ref, xw_ref, dv_ref, b1_ref, w2_ref,
             xw2_ref, t3_ref):
        dv = dv_ref[...]
        agg = jnp.concatenate([aa_ref[0], ab_ref[0]], axis=1)
        h1 = dv * agg + (dv * dv) * xw_ref[...] + b1_ref[...]
        h1 = jnp.maximum(h1, 0.0)
        xw2 = jnp.dot(h1, w2_ref[...], preferred_element_type=jnp.float32)
        xw2_ref[...] = xw2
        t2 = xw2 * dv
        t3_ref[...] = jnp.stack([t2[:, :16], t2[:, 16:]], axis=0)
    return pl.pallas_call(
        body,
        grid=(_TG,),
        in_specs=[
            pl.BlockSpec((1, _R, 32), lambda i: (0, i, 0)),
            pl.BlockSpec((1, _R, 32), lambda i: (1, i, 0)),
            pl.BlockSpec((_R, 64), lambda i: (i, 0)),
            pl.BlockSpec((_R, 1), lambda i: (i, 0)),
            pl.BlockSpec((1, 64), lambda i: (0, 0)),
            pl.BlockSpec((64, 32), lambda i: (0, 0)),
        ],
        out_specs=[
            pl.BlockSpec((_R, 32), lambda i: (i, 0)),
            pl.BlockSpec((2, _R, 16), lambda i: (0, i, 0)),
        ],
        out_shape=[
            jax.ShapeDtypeStruct((N, 32), jnp.float32),
            jax.ShapeDtypeStruct((2, N, 16), jnp.float32),
        ],
    )(agg3, agg3, xw1, dinv, b1.reshape(1, 64), W2)


def _tc_conv2_post(agg3, xw2, dinv, b2):
    def body(aa_ref, ab_ref, xw_ref, dv_ref, b2_ref, o_ref):
        dv = dv_ref[...]
        agg = jnp.concatenate([aa_ref[0], ab_ref[0]], axis=1)
        h2 = dv * agg + (dv * dv) * xw_ref[...] + b2_ref[...]
        o_ref[...] = h2.astype(jnp.bfloat16)
    return pl.pallas_call(
        body,
        grid=(_TG,),
        in_specs=[
            pl.BlockSpec((1, _R, 16), lambda i: (0, i, 0)),
            pl.BlockSpec((1, _R, 16), lambda i: (1, i, 0)),
            pl.BlockSpec((_R, 32), lambda i: (i, 0)),
            pl.BlockSpec((_R, 1), lambda i: (i, 0)),
            pl.BlockSpec((1, 32), lambda i: (0, 0)),
        ],
        out_specs=pl.BlockSpec((_R, 32), lambda i: (i, 0)),
        out_shape=jax.ShapeDtypeStruct((N, 32), jnp.bfloat16),
    )(agg3, agg3, xw2, dinv, b2.reshape(1, 32))


def _tc_mlp(efsp, Wp1, bp1, Wp2, bp2):
    EP4 = ELP // 4
    R2 = 1792
    eye4 = jnp.eye(4, dtype=jnp.float32)
    W1b = jnp.kron(eye4, Wp1)
    b1b = jnp.tile(bp1, 4).reshape(1, 64)
    W2b = jnp.kron(eye4, Wp2)
    b2b = jnp.tile(bp2, 4).reshape(1, 4)

    def body(ef_ref, w1_ref, b1_ref, w2_ref, b2_ref, o_ref):
        ef = ef_ref[...].astype(jnp.float32)
        e = jnp.dot(ef, w1_ref[...],
                    preferred_element_type=jnp.float32) + b1_ref[...]
        e = jnp.maximum(e, 0.0)
        z = jnp.dot(e, w2_ref[...],
                    preferred_element_type=jnp.float32) + b2_ref[...]
        o_ref[...] = 1.0 / (1.0 + jnp.exp(-z))
    return pl.pallas_call(
        body,
        grid=(EP4 // R2,),
        in_specs=[
            pl.BlockSpec((R2, 128), lambda i: (i, 0)),
            pl.BlockSpec((128, 64), lambda i: (0, 0)),
            pl.BlockSpec((1, 64), lambda i: (0, 0)),
            pl.BlockSpec((64, 4), lambda i: (0, 0)),
            pl.BlockSpec((1, 4), lambda i: (0, 0)),
        ],
        out_specs=pl.BlockSpec((R2, 4), lambda i: (i, 0)),
        out_shape=jax.ShapeDtypeStruct((EP4, 4), jnp.float32),
    )(efsp, W1b, b1b, W2b, b2b)


def kernel(x, edge_index, edge_label_index,
           W_enc, b_enc, W1, b1, W2, b2, Wp1, bp1, Wp2, bp2):
    f32 = jnp.float32
    i32 = jnp.int32

    src = edge_index[0]
    dst = edge_index[1]
    src_p = jnp.concatenate(
        [src, jnp.zeros((EP - E,), i32)]).reshape(EP // CHUNK, CHUNK)
    dst_p = jnp.concatenate(
        [dst, jnp.full((EP - E,), N, i32)]).reshape(EP // CHUNK, CHUNK)
    lsrc_p = jnp.concatenate(
        [edge_label_index[0], jnp.zeros((ELP - EL,), i32)]
    ).reshape(ELP // CHUNK, CHUNK)
    ldst_p = jnp.concatenate(
        [edge_label_index[1], jnp.zeros((ELP - EL,), i32)]
    ).reshape(ELP // CHUNK, CHUNK)

    z1 = jnp.zeros((APT,), f32)
    z32 = jnp.zeros((ZC, 32), f32)
    z16 = jnp.zeros((ZC, 16), f32)
    ones = jnp.ones((CHUNK,), f32)

    degp = _deg_kernel(dst_p, ones, z1)
    degp_t = degp[:, :N].T
    xw1, dinv, t3 = _tc_encoder(x, degp_t, W_enc, b_enc, W1)
    agg3 = _conv_agg(t3, src_p, dst_p, z32, 32)
    xw2, t32 = _tc_conv1_post(agg3, xw1, dinv, b1, W2)
    agg23 = _conv_agg(t32, src_p, dst_p, z16, 16)
    h2 = _tc_conv2_post(agg23, xw2, dinv, b2)
    efp = _label_gather(h2, lsrc_p, ldst_p)
    out = _tc_mlp(efp, Wp1, bp1, Wp2, bp2)
    return out.reshape(ELP)[:EL]


# speedup vs baseline: 2.2920x; 1.0488x over previous
import functools

import jax
import jax.numpy as jnp
from jax import lax
from jax.experimental import pallas as pl
from jax.experimental.pallas import tpu as pltpu
from jax.experimental.pallas import tpu_sc as plsc

N = 50000
E = 800000
EL = 200000
NC, NS = 2, 16
NW = NC * NS
CHUNK = 256
KE = 98
EP = NW * KE * CHUNK
IB = 7
KE0, KE1 = 98, 98
KL0, KL1 = 21, 28
ELP = NS * (KL0 + KL1) * CHUNK
LB = 7
ACC_ROWS = 50176
APT = ACC_ROWS // NS
ZC = 112


def _sc_mesh():
    return plsc.VectorSubcoreMesh(
        core_axis_name="c", subcore_axis_name="s",
        num_cores=NC, num_subcores=NS)


_SC_PARAMS = pltpu.CompilerParams(use_tc_tiling_on_sc=False)


def _deg_kernel(dst2d, ones, zrows):
    @functools.partial(
        pl.kernel,
        out_type=jax.ShapeDtypeStruct((NC, ACC_ROWS), jnp.float32),
        mesh=_sc_mesh(),
        compiler_params=_SC_PARAMS,
        scratch_types=[
            pltpu.VMEM((KE, CHUNK), jnp.int32),
            pltpu.VMEM((CHUNK,), jnp.float32),
            pltpu.VMEM((APT,), jnp.float32),
            pltpu.VMEM_SHARED((ACC_ROWS,), jnp.float32),
        ],
    )
    def deg(dst_hbm, ones_hbm, z_hbm, out_hbm, idx_v, ones_v, zbuf, acc):
        c = lax.axis_index("c")
        s = lax.axis_index("s")
        wid = c * NS + s
        sl = pl.ds(s * APT, APT)
        pltpu.sync_copy(z_hbm, zbuf)
        pltpu.sync_copy(zbuf, acc.at[sl])
        pltpu.sync_copy(ones_hbm, ones_v)
        pltpu.sync_copy(dst_hbm.at[pl.ds(wid * KE, KE)], idx_v)
        plsc.subcore_barrier()

        def body(j, carry):
            pltpu.sync_copy(ones_v, acc.at[idx_v.at[j]], add=True)
            return carry
        lax.fori_loop(0, KE, body, 0)
        plsc.subcore_barrier()
        pltpu.sync_copy(acc.at[sl], zbuf)
        pltpu.sync_copy(zbuf, out_hbm.at[c, sl])

    return deg(dst2d, ones, zrows)


def _conv_agg(t3, src2d, dst2d, zrows, F, CH, KA, KB):
    NZ = APT // ZC

    @functools.partial(
        pl.kernel,
        out_type=jax.ShapeDtypeStruct((NC, ACC_ROWS, F), jnp.float32),
        mesh=_sc_mesh(),
        compiler_params=_SC_PARAMS,
        scratch_types=[
            pltpu.VMEM((IB, CH), jnp.int32),
            pltpu.VMEM((IB, CH), jnp.int32),
            pltpu.VMEM((CH, F), jnp.float32),
            pltpu.VMEM((CH, F), jnp.float32),
            pltpu.VMEM((ZC, F), jnp.float32),
            pltpu.SemaphoreType.DMA,
            pltpu.SemaphoreType.DMA,
            pltpu.SemaphoreType.DMA,
            pltpu.SemaphoreType.DMA,
            pltpu.VMEM_SHARED((ACC_ROWS, F), jnp.float32),
        ],
    )
    def agg(t3_hbm, src_hbm, dst_hbm, z_hbm, out_hbm,
            src_v, dst_v, rows0, rows1, zstage, sg0, sg1, ss0, ss1, acc):
        c = lax.axis_index("c")
        s = lax.axis_index("s")
        tbl = t3_hbm.at[c]
        rows = (rows0, rows1)
        sg = (sg0, sg1)
        ss = (ss0, ss1)
        pltpu.sync_copy(z_hbm, zstage)
        for k in range(NZ):
            pltpu.async_copy(zstage, acc.at[pl.ds(s * APT + k * ZC, ZC)],
                             sg0)
        for k in range(NZ):
            pltpu.make_async_copy(
                zstage, acc.at[pl.ds(s * APT + k * ZC, ZC)], sg0).wait()
        plsc.subcore_barrier()

        base = jnp.where(c == 0, s * KA, NS * KA + s * KB)
        ng = jnp.where(c == 0, KA // IB, KB // IB)

        def outer(g, carry):
            gb = base + g * IB
            pltpu.sync_copy(src_hbm.at[pl.ds(gb, IB)], src_v)
            pltpu.sync_copy(dst_hbm.at[pl.ds(gb, IB)], dst_v)
            dg = [None, None]
            dsc = [None, None]
            dg[0] = pltpu.async_copy(tbl.at[src_v.at[0]], rows[0], sg[0])
            for k in range(IB):
                b = k % 2
                nb = 1 - b
                dg[b].wait()
                if k + 1 < IB:
                    if dsc[nb] is not None:
                        dsc[nb].wait()
                    dg[nb] = pltpu.async_copy(
                        tbl.at[src_v.at[k + 1]], rows[nb], sg[nb])
                dsc[b] = pltpu.async_copy(
                    rows[b], acc.at[dst_v.at[k]], ss[b], add=True)
            dsc[0].wait()
            dsc[1].wait()
            return carry
        lax.fori_loop(0, ng, outer, 0)
        plsc.subcore_barrier()

        stg = (rows0.at[pl.ds(0, ZC)], rows1.at[pl.ds(0, ZC)])
        dst_done = [None, None]
        for k in range(NZ):
            b = k % 2
            sk = pl.ds(s * APT + k * ZC, ZC)
            if dst_done[b] is not None:
                dst_done[b].wait()
            pltpu.async_copy(acc.at[sk], stg[b], sg[b]).wait()
            dst_done[b] = pltpu.async_copy(stg[b], out_hbm.at[c, sk], ss[b])
        dst_done[0].wait()
        dst_done[1].wait()

    return agg(t3, src2d, dst2d, zrows)


def _label_gather(h2, lsrc2d, ldst2d):
    KLM = max(KL0, KL1)
    C4 = CHUNK // 4

    @functools.partial(
        pl.kernel,
        out_type=jax.ShapeDtypeStruct((ELP // 4, 128), jnp.bfloat16),
        mesh=_sc_mesh(),
        compiler_params=_SC_PARAMS,
        scratch_types=[
            pltpu.VMEM((KLM, CHUNK), jnp.int32),
            pltpu.VMEM((KLM, CHUNK), jnp.int32),
            pltpu.VMEM((CHUNK, 32), jnp.bfloat16),
            pltpu.VMEM((CHUNK, 32), jnp.bfloat16),
            pltpu.VMEM((CHUNK, 32), jnp.bfloat16),
            pltpu.VMEM((CHUNK, 32), jnp.bfloat16),
            pltpu.VMEM((C4, 128), jnp.bfloat16),
            pltpu.VMEM((C4, 128), jnp.bfloat16),
            pltpu.SemaphoreType.DMA,
            pltpu.SemaphoreType.DMA,
            pltpu.SemaphoreType.DMA,
            pltpu.SemaphoreType.DMA,
            pltpu.SemaphoreType.DMA,
            pltpu.SemaphoreType.DMA,
        ],
    )
    def lab(h2_hbm, src_hbm, dst_hbm, ef_hbm, src_v, dst_v,
            bs0, bs1, bd0, bd1, bp0, bp1, gs0, gs1, gd0, gd1, ws0, ws1):
        c = lax.axis_index("c")
        s = lax.axis_index("s")
        base = jnp.where(c == 0, s * KL0, NS * KL0 + s * KL1)
        ng = jnp.where(c == 0, KL0 // LB, KL1 // LB)

        @pl.when(c == 0)
        def _():
            pltpu.sync_copy(src_hbm.at[pl.ds(s * KL0, KL0)],
                            src_v.at[pl.ds(0, KL0)])
            pltpu.sync_copy(dst_hbm.at[pl.ds(s * KL0, KL0)],
                            dst_v.at[pl.ds(0, KL0)])

        @pl.when(c == 1)
        def _():
            pltpu.sync_copy(src_hbm.at[pl.ds(NS * KL0 + s * KL1, KL1)],
                            src_v.at[pl.ds(0, KL1)])
            pltpu.sync_copy(dst_hbm.at[pl.ds(NS * KL0 + s * KL1, KL1)],
                            dst_v.at[pl.ds(0, KL1)])

        bufs_s = (bs0, bs1)
        bufs_d = (bd0, bd1)
        bufs_p = (bp0, bp1)
        gs = (gs0, gs1)
        gd = (gd0, gd1)
        ws = (ws0, ws1)

        def outer(g, carry):
            jb = g * LB
            dgs = [None, None]
            dgd = [None, None]
            dsp = [None, None]
            dgs[0] = pltpu.async_copy(
                h2_hbm.at[src_v.at[jb]], bufs_s[0], gs[0])
            dgd[0] = pltpu.async_copy(
                h2_hbm.at[dst_v.at[jb]], bufs_d[0], gd[0])
            for k in range(LB):
                b = k % 2
                nb = 1 - b
                j = jb + k
                dgs[b].wait()
                dgd[b].wait()
                if k + 1 < LB:
                    if dsp[nb] is not None:
                        dsp[nb].wait()
                    dgs[nb] = pltpu.async_copy(
                        h2_hbm.at[src_v.at[j + 1]], bufs_s[nb], gs[nb])
                    dgd[nb] = pltpu.async_copy(
                        h2_hbm.at[dst_v.at[j + 1]], bufs_d[nb], gd[nb])
                bsb, bdb, bpb = bufs_s[b], bufs_d[b], bufs_p[b]

                def repack(q, carry2):
                    for sub in range(4):
                        v = bsb[4 * q + sub, :] + bdb[4 * q + sub, :]
                        bpb[q, pl.ds(sub * 32, 32)] = v
                    return carry2
                lax.fori_loop(0, C4, repack, 0)
                dsp[b] = pltpu.async_copy(
                    bpb, ef_hbm.at[pl.ds((base + j) * C4, C4)], ws[b])
            dsp[0].wait()
            dsp[1].wait()
            return carry
        lax.fori_loop(0, ng, outer, 0)

    return lab(h2, lsrc2d, ldst2d)


_R = 1024
_TG = ACC_ROWS // _R


def _tc_encoder(x, degp_t, W_enc, b_enc, W1):
    def body(x_ref, dp_ref, we_ref, be_ref, w1_ref,
             xw1_ref, dinv_ref, t3_ref):
        h = jnp.dot(x_ref[...], we_ref[...],
                    preferred_element_type=jnp.float32) + be_ref[...]
        h = jnp.maximum(h, 0.0)
        xw1 = jnp.dot(h, w1_ref[...], preferred_element_type=jnp.float32)
        xw1_ref[...] = xw1
        deg = dp_ref[...][:, 0:1] + dp_ref[...][:, 1:2] + 1.0
        dinv = lax.rsqrt(deg)
        dinv_ref[...] = dinv
        t = xw1 * dinv
        t3_ref[...] = jnp.stack([t[:, :32], t[:, 32:]], axis=0)
    return pl.pallas_call(
        body,
        grid=(_TG,),
        in_specs=[
            pl.BlockSpec((_R, 128), lambda i: (i, 0)),
            pl.BlockSpec((_R, 2), lambda i: (i, 0)),
            pl.BlockSpec((128, 64), lambda i: (0, 0)),
            pl.BlockSpec((1, 64), lambda i: (0, 0)),
            pl.BlockSpec((64, 64), lambda i: (0, 0)),
        ],
        out_specs=[
            pl.BlockSpec((_R, 64), lambda i: (i, 0)),
            pl.BlockSpec((_R, 1), lambda i: (i, 0)),
            pl.BlockSpec((2, _R, 32), lambda i: (0, i, 0)),
        ],
        out_shape=[
            jax.ShapeDtypeStruct((N, 64), jnp.float32),
            jax.ShapeDtypeStruct((N, 1), jnp.float32),
            jax.ShapeDtypeStruct((2, N, 32), jnp.float32),
        ],
    )(x, degp_t, W_enc, b_enc.reshape(1, 64), W1)


def _tc_conv1_post(agg3, xw1, dinv, b1, W2):
    def body(aa_ref, ab_ref, xw_ref, dv_ref, b1_ref, w2_ref,
             xw2_ref, t3_ref):
        dv = dv_ref[...]
        agg = jnp.concatenate([aa_ref[0], ab_ref[0]], axis=1)
        h1 = dv * agg + (dv * dv) * xw_ref[...] + b1_ref[...]
        h1 = jnp.maximum(h1, 0.0)
        xw2 = jnp.dot(h1, w2_ref[...], preferred_element_type=jnp.float32)
        xw2_ref[...] = xw2
        t2 = xw2 * dv
        t3_ref[...] = jnp.stack([t2[:, :16], t2[:, 16:]], axis=0)
    return pl.pallas_call(
        body,
        grid=(_TG,),
        in_specs=[
            pl.BlockSpec((1, _R, 32), lambda i: (0, i, 0)),
            pl.BlockSpec((1, _R, 32), lambda i: (1, i, 0)),
            pl.BlockSpec((_R, 64), lambda i: (i, 0)),
            pl.BlockSpec((_R, 1), lambda i: (i, 0)),
            pl.BlockSpec((1, 64), lambda i: (0, 0)),
            pl.BlockSpec((64, 32), lambda i: (0, 0)),
        ],
        out_specs=[
            pl.BlockSpec((_R, 32), lambda i: (i, 0)),
            pl.BlockSpec((2, _R, 16), lambda i: (0, i, 0)),
        ],
        out_shape=[
            jax.ShapeDtypeStruct((N, 32), jnp.float32),
            jax.ShapeDtypeStruct((2, N, 16), jnp.float32),
        ],
    )(agg3, agg3, xw1, dinv, b1.reshape(1, 64), W2)


def _tc_conv2_post(agg3, xw2, dinv, b2):
    def body(aa_ref, ab_ref, xw_ref, dv_ref, b2_ref, o_ref):
        dv = dv_ref[...]
        agg = jnp.concatenate([aa_ref[0], ab_ref[0]], axis=1)
        h2 = dv * agg + (dv * dv) * xw_ref[...] + b2_ref[...]
        o_ref[...] = h2.astype(jnp.bfloat16)
    return pl.pallas_call(
        body,
        grid=(_TG,),
        in_specs=[
            pl.BlockSpec((1, _R, 16), lambda i: (0, i, 0)),
            pl.BlockSpec((1, _R, 16), lambda i: (1, i, 0)),
            pl.BlockSpec((_R, 32), lambda i: (i, 0)),
            pl.BlockSpec((_R, 1), lambda i: (i, 0)),
            pl.BlockSpec((1, 32), lambda i: (0, 0)),
        ],
        out_specs=pl.BlockSpec((_R, 32), lambda i: (i, 0)),
        out_shape=jax.ShapeDtypeStruct((N, 32), jnp.bfloat16),
    )(agg3, agg3, xw2, dinv, b2.reshape(1, 32))


def _tc_mlp(efsp, Wp1, bp1, Wp2, bp2):
    EP4 = ELP // 4
    R2 = 1792
    eye4 = jnp.eye(4, dtype=jnp.float32)
    W1b = jnp.kron(eye4, Wp1)
    b1b = jnp.tile(bp1, 4).reshape(1, 64)
    W2b = jnp.kron(eye4, Wp2)
    b2b = jnp.tile(bp2, 4).reshape(1, 4)

    def body(ef_ref, w1_ref, b1_ref, w2_ref, b2_ref, o_ref):
        ef = ef_ref[...].astype(jnp.float32)
        e = jnp.dot(ef, w1_ref[...],
                    preferred_element_type=jnp.float32) + b1_ref[...]
        e = jnp.maximum(e, 0.0)
        z = jnp.dot(e, w2_ref[...],
                    preferred_element_type=jnp.float32) + b2_ref[...]
        o_ref[...] = 1.0 / (1.0 + jnp.exp(-z))
    return pl.pallas_call(
        body,
        grid=(EP4 // R2,),
        in_specs=[
            pl.BlockSpec((R2, 128), lambda i: (i, 0)),
            pl.BlockSpec((128, 64), lambda i: (0, 0)),
            pl.BlockSpec((1, 64), lambda i: (0, 0)),
            pl.BlockSpec((64, 4), lambda i: (0, 0)),
            pl.BlockSpec((1, 4), lambda i: (0, 0)),
        ],
        out_specs=pl.BlockSpec((R2, 4), lambda i: (i, 0)),
        out_shape=jax.ShapeDtypeStruct((EP4, 4), jnp.float32),
    )(efsp, W1b, b1b, W2b, b2b)


def kernel(x, edge_index, edge_label_index,
           W_enc, b_enc, W1, b1, W2, b2, Wp1, bp1, Wp2, bp2):
    f32 = jnp.float32
    i32 = jnp.int32

    src = edge_index[0]
    dst = edge_index[1]
    src_p = jnp.concatenate(
        [src, jnp.zeros((EP - E,), i32)]).reshape(EP // CHUNK, CHUNK)
    dst_p = jnp.concatenate(
        [dst, jnp.full((EP - E,), N, i32)]).reshape(EP // CHUNK, CHUNK)
    lsrc_p = jnp.concatenate(
        [edge_label_index[0], jnp.zeros((ELP - EL,), i32)]
    ).reshape(ELP // CHUNK, CHUNK)
    ldst_p = jnp.concatenate(
        [edge_label_index[1], jnp.zeros((ELP - EL,), i32)]
    ).reshape(ELP // CHUNK, CHUNK)

    z1 = jnp.zeros((APT,), f32)
    z32 = jnp.zeros((ZC, 32), f32)
    z16 = jnp.zeros((ZC, 16), f32)
    ones = jnp.ones((CHUNK,), f32)

    degp = _deg_kernel(dst_p, ones, z1)
    degp_t = degp[:, :N].T
    xw1, dinv, t3 = _tc_encoder(x, degp_t, W_enc, b_enc, W1)
    agg3 = _conv_agg(t3, src_p, dst_p, z32, 32, CHUNK, KE0, KE1)
    xw2, t32 = _tc_conv1_post(agg3, xw1, dinv, b1, W2)
    src_p2 = src_p.reshape(EP // (2 * CHUNK), 2 * CHUNK)
    dst_p2 = dst_p.reshape(EP // (2 * CHUNK), 2 * CHUNK)
    agg23 = _conv_agg(t32, src_p2, dst_p2, z16, 16, 2 * CHUNK,
                      KE0 // 2, KE1 // 2)
    h2 = _tc_conv2_post(agg23, xw2, dinv, b2)
    efp = _label_gather(h2, lsrc_p, ldst_p)
    out = _tc_mlp(efp, Wp1, bp1, Wp2, bp2)
    return out.reshape(ELP)[:EL]


# speedup vs baseline: 2.3128x; 1.0091x over previous
import functools

import jax
import jax.numpy as jnp
from jax import lax
from jax.experimental import pallas as pl
from jax.experimental.pallas import tpu as pltpu
from jax.experimental.pallas import tpu_sc as plsc

N = 50000
E = 800000
EL = 200000
NC, NS = 2, 16
NW = NC * NS
CHUNK = 256
KE = 98
EP = NW * KE * CHUNK
KE0, KE1 = 98, 98
KL0, KL1 = 21, 28
ELP = NS * (KL0 + KL1) * CHUNK
LB = 7
ACC_ROWS = 50176
APT = ACC_ROWS // NS
ZC = 112


def _sc_mesh():
    return plsc.VectorSubcoreMesh(
        core_axis_name="c", subcore_axis_name="s",
        num_cores=NC, num_subcores=NS)


_SC_PARAMS = pltpu.CompilerParams(use_tc_tiling_on_sc=False)


def _deg_kernel(dst2d, ones, zrows):
    @functools.partial(
        pl.kernel,
        out_type=jax.ShapeDtypeStruct((NC, ACC_ROWS), jnp.float32),
        mesh=_sc_mesh(),
        compiler_params=_SC_PARAMS,
        scratch_types=[
            pltpu.VMEM((KE, CHUNK), jnp.int32),
            pltpu.VMEM((CHUNK,), jnp.float32),
            pltpu.VMEM((APT,), jnp.float32),
            pltpu.VMEM_SHARED((ACC_ROWS,), jnp.float32),
        ],
    )
    def deg(dst_hbm, ones_hbm, z_hbm, out_hbm, idx_v, ones_v, zbuf, acc):
        c = lax.axis_index("c")
        s = lax.axis_index("s")
        wid = c * NS + s
        sl = pl.ds(s * APT, APT)
        pltpu.sync_copy(z_hbm, zbuf)
        pltpu.sync_copy(zbuf, acc.at[sl])
        pltpu.sync_copy(ones_hbm, ones_v)
        pltpu.sync_copy(dst_hbm.at[pl.ds(wid * KE, KE)], idx_v)
        plsc.subcore_barrier()

        def body(j, carry):
            pltpu.sync_copy(ones_v, acc.at[idx_v.at[j]], add=True)
            return carry
        lax.fori_loop(0, KE, body, 0)
        plsc.subcore_barrier()
        pltpu.sync_copy(acc.at[sl], zbuf)
        pltpu.sync_copy(zbuf, out_hbm.at[c, sl])

    return deg(dst2d, ones, zrows)


def _conv_agg(t3, src2d, dst2d, zrows, F, CH, KA, KB, IB):
    NZ = APT // ZC

    @functools.partial(
        pl.kernel,
        out_type=jax.ShapeDtypeStruct((NC, ACC_ROWS, F), jnp.float32),
        mesh=_sc_mesh(),
        compiler_params=_SC_PARAMS,
        scratch_types=[
            pltpu.VMEM((IB, CH), jnp.int32),
            pltpu.VMEM((IB, CH), jnp.int32),
            pltpu.VMEM((CH, F), jnp.float32),
            pltpu.VMEM((CH, F), jnp.float32),
            pltpu.VMEM((ZC, F), jnp.float32),
            pltpu.SemaphoreType.DMA,
            pltpu.SemaphoreType.DMA,
            pltpu.SemaphoreType.DMA,
            pltpu.SemaphoreType.DMA,
            pltpu.VMEM_SHARED((ACC_ROWS, F), jnp.float32),
        ],
    )
    def agg(t3_hbm, src_hbm, dst_hbm, z_hbm, out_hbm,
            src_v, dst_v, rows0, rows1, zstage, sg0, sg1, ss0, ss1, acc):
        c = lax.axis_index("c")
        s = lax.axis_index("s")
        tbl = t3_hbm.at[c]
        rows = (rows0, rows1)
        sg = (sg0, sg1)
        ss = (ss0, ss1)
        pltpu.sync_copy(z_hbm, zstage)
        for k in range(NZ):
            pltpu.async_copy(zstage, acc.at[pl.ds(s * APT + k * ZC, ZC)],
                             sg0)
        for k in range(NZ):
            pltpu.make_async_copy(
                zstage, acc.at[pl.ds(s * APT + k * ZC, ZC)], sg0).wait()
        plsc.subcore_barrier()

        base = jnp.where(c == 0, s * KA, NS * KA + s * KB)
        ng = jnp.where(c == 0, KA // IB, KB // IB)

        def outer(g, carry):
            gb = base + g * IB
            pltpu.sync_copy(src_hbm.at[pl.ds(gb, IB)], src_v)
            pltpu.sync_copy(dst_hbm.at[pl.ds(gb, IB)], dst_v)
            dg = [None, None]
            dsc = [None, None]
            dg[0] = pltpu.async_copy(tbl.at[src_v.at[0]], rows[0], sg[0])
            for k in range(IB):
                b = k % 2
                nb = 1 - b
                dg[b].wait()
                if k + 1 < IB:
                    if dsc[nb] is not None:
                        dsc[nb].wait()
                    dg[nb] = pltpu.async_copy(
                        tbl.at[src_v.at[k + 1]], rows[nb], sg[nb])
                dsc[b] = pltpu.async_copy(
                    rows[b], acc.at[dst_v.at[k]], ss[b], add=True)
            dsc[0].wait()
            dsc[1].wait()
            return carry
        lax.fori_loop(0, ng, outer, 0)
        plsc.subcore_barrier()

        stg = (rows0.at[pl.ds(0, ZC)], rows1.at[pl.ds(0, ZC)])
        dst_done = [None, None]
        for k in range(NZ):
            b = k % 2
            sk = pl.ds(s * APT + k * ZC, ZC)
            if dst_done[b] is not None:
                dst_done[b].wait()
            pltpu.async_copy(acc.at[sk], stg[b], sg[b]).wait()
            dst_done[b] = pltpu.async_copy(stg[b], out_hbm.at[c, sk], ss[b])
        dst_done[0].wait()
        dst_done[1].wait()

    return agg(t3, src2d, dst2d, zrows)


def _label_gather(h2, lsrc2d, ldst2d):
    KLM = max(KL0, KL1)
    C4 = CHUNK // 4

    @functools.partial(
        pl.kernel,
        out_type=jax.ShapeDtypeStruct((ELP // 4, 128), jnp.bfloat16),
        mesh=_sc_mesh(),
        compiler_params=_SC_PARAMS,
        scratch_types=[
            pltpu.VMEM((KLM, CHUNK), jnp.int32),
            pltpu.VMEM((KLM, CHUNK), jnp.int32),
            pltpu.VMEM((CHUNK, 32), jnp.bfloat16),
            pltpu.VMEM((CHUNK, 32), jnp.bfloat16),
            pltpu.VMEM((CHUNK, 32), jnp.bfloat16),
            pltpu.VMEM((CHUNK, 32), jnp.bfloat16),
            pltpu.VMEM((C4, 128), jnp.bfloat16),
            pltpu.VMEM((C4, 128), jnp.bfloat16),
            pltpu.SemaphoreType.DMA,
            pltpu.SemaphoreType.DMA,
            pltpu.SemaphoreType.DMA,
            pltpu.SemaphoreType.DMA,
            pltpu.SemaphoreType.DMA,
            pltpu.SemaphoreType.DMA,
        ],
    )
    def lab(h2_hbm, src_hbm, dst_hbm, ef_hbm, src_v, dst_v,
            bs0, bs1, bd0, bd1, bp0, bp1, gs0, gs1, gd0, gd1, ws0, ws1):
        c = lax.axis_index("c")
        s = lax.axis_index("s")
        base = jnp.where(c == 0, s * KL0, NS * KL0 + s * KL1)
        ng = jnp.where(c == 0, KL0 // LB, KL1 // LB)

        @pl.when(c == 0)
        def _():
            pltpu.sync_copy(src_hbm.at[pl.ds(s * KL0, KL0)],
                            src_v.at[pl.ds(0, KL0)])
            pltpu.sync_copy(dst_hbm.at[pl.ds(s * KL0, KL0)],
                            dst_v.at[pl.ds(0, KL0)])

        @pl.when(c == 1)
        def _():
            pltpu.sync_copy(src_hbm.at[pl.ds(NS * KL0 + s * KL1, KL1)],
                            src_v.at[pl.ds(0, KL1)])
            pltpu.sync_copy(dst_hbm.at[pl.ds(NS * KL0 + s * KL1, KL1)],
                            dst_v.at[pl.ds(0, KL1)])

        bufs_s = (bs0, bs1)
        bufs_d = (bd0, bd1)
        bufs_p = (bp0, bp1)
        gs = (gs0, gs1)
        gd = (gd0, gd1)
        ws = (ws0, ws1)

        def outer(g, carry):
            jb = g * LB
            dgs = [None, None]
            dgd = [None, None]
            dsp = [None, None]
            dgs[0] = pltpu.async_copy(
                h2_hbm.at[src_v.at[jb]], bufs_s[0], gs[0])
            dgd[0] = pltpu.async_copy(
                h2_hbm.at[dst_v.at[jb]], bufs_d[0], gd[0])
            for k in range(LB):
                b = k % 2
                nb = 1 - b
                j = jb + k
                dgs[b].wait()
                dgd[b].wait()
                if k + 1 < LB:
                    if dsp[nb] is not None:
                        dsp[nb].wait()
                    dgs[nb] = pltpu.async_copy(
                        h2_hbm.at[src_v.at[j + 1]], bufs_s[nb], gs[nb])
                    dgd[nb] = pltpu.async_copy(
                        h2_hbm.at[dst_v.at[j + 1]], bufs_d[nb], gd[nb])
                bsb, bdb, bpb = bufs_s[b], bufs_d[b], bufs_p[b]

                def repack(q, carry2):
                    for sub in range(4):
                        v = bsb[4 * q + sub, :] + bdb[4 * q + sub, :]
                        bpb[q, pl.ds(sub * 32, 32)] = v
                    return carry2
                lax.fori_loop(0, C4, repack, 0)
                dsp[b] = pltpu.async_copy(
                    bpb, ef_hbm.at[pl.ds((base + j) * C4, C4)], ws[b])
            dsp[0].wait()
            dsp[1].wait()
            return carry
        lax.fori_loop(0, ng, outer, 0)

    return lab(h2, lsrc2d, ldst2d)


_R = 1024
_TG = ACC_ROWS // _R


def _tc_encoder(x, degp_t, W_enc, b_enc, W1):
    def body(x_ref, dp_ref, we_ref, be_ref, w1_ref,
             xw1_ref, dinv_ref, t3_ref):
        h = jnp.dot(x_ref[...], we_ref[...],
                    preferred_element_type=jnp.float32) + be_ref[...]
        h = jnp.maximum(h, 0.0)
        xw1 = jnp.dot(h, w1_ref[...], preferred_element_type=jnp.float32)
        xw1_ref[...] = xw1
        deg = dp_ref[...][:, 0:1] + dp_ref[...][:, 1:2] + 1.0
        dinv = lax.rsqrt(deg)
        dinv_ref[...] = dinv
        t = xw1 * dinv
        t3_ref[...] = jnp.stack([t[:, :32], t[:, 32:]], axis=0)
    return pl.pallas_call(
        body,
        grid=(_TG,),
        in_specs=[
            pl.BlockSpec((_R, 128), lambda i: (i, 0)),
            pl.BlockSpec((_R, 2), lambda i: (i, 0)),
            pl.BlockSpec((128, 64), lambda i: (0, 0)),
            pl.BlockSpec((1, 64), lambda i: (0, 0)),
            pl.BlockSpec((64, 64), lambda i: (0, 0)),
        ],
        out_specs=[
            pl.BlockSpec((_R, 64), lambda i: (i, 0)),
            pl.BlockSpec((_R, 1), lambda i: (i, 0)),
            pl.BlockSpec((2, _R, 32), lambda i: (0, i, 0)),
        ],
        out_shape=[
            jax.ShapeDtypeStruct((N, 64), jnp.float32),
            jax.ShapeDtypeStruct((N, 1), jnp.float32),
            jax.ShapeDtypeStruct((2, N, 32), jnp.float32),
        ],
    )(x, degp_t, W_enc, b_enc.reshape(1, 64), W1)


def _tc_conv1_post(agg3, xw1, dinv, b1, W2):
    def body(aa_ref, ab_ref, xw_ref, dv_ref, b1_ref, w2_ref,
             xw2_ref, t3_ref):
        dv = dv_ref[...]
        agg = jnp.concatenate([aa_ref[0], ab_ref[0]], axis=1)
        h1 = dv * agg + (dv * dv) * xw_ref[...] + b1_ref[...]
        h1 = jnp.maximum(h1, 0.0)
        xw2 = jnp.dot(h1, w2_ref[...], preferred_element_type=jnp.float32)
        xw2_ref[...] = xw2
        t2 = xw2 * dv
        t3_ref[...] = jnp.stack([t2[:, :16], t2[:, 16:]], axis=0)
    return pl.pallas_call(
        body,
        grid=(_TG,),
        in_specs=[
            pl.BlockSpec((1, _R, 32), lambda i: (0, i, 0)),
            pl.BlockSpec((1, _R, 32), lambda i: (1, i, 0)),
            pl.BlockSpec((_R, 64), lambda i: (i, 0)),
            pl.BlockSpec((_R, 1), lambda i: (i, 0)),
            pl.BlockSpec((1, 64), lambda i: (0, 0)),
            pl.BlockSpec((64, 32), lambda i: (0, 0)),
        ],
        out_specs=[
            pl.BlockSpec((_R, 32), lambda i: (i, 0)),
            pl.BlockSpec((2, _R, 16), lambda i: (0, i, 0)),
        ],
        out_shape=[
            jax.ShapeDtypeStruct((N, 32), jnp.float32),
            jax.ShapeDtypeStruct((2, N, 16), jnp.float32),
        ],
    )(agg3, agg3, xw1, dinv, b1.reshape(1, 64), W2)


def _tc_conv2_post(agg3, xw2, dinv, b2):
    def body(aa_ref, ab_ref, xw_ref, dv_ref, b2_ref, o_ref):
        dv = dv_ref[...]
        agg = jnp.concatenate([aa_ref[0], ab_ref[0]], axis=1)
        h2 = dv * agg + (dv * dv) * xw_ref[...] + b2_ref[...]
        o_ref[...] = h2.astype(jnp.bfloat16)
    return pl.pallas_call(
        body,
        grid=(_TG,),
        in_specs=[
            pl.BlockSpec((1, _R, 16), lambda i: (0, i, 0)),
            pl.BlockSpec((1, _R, 16), lambda i: (1, i, 0)),
            pl.BlockSpec((_R, 32), lambda i: (i, 0)),
            pl.BlockSpec((_R, 1), lambda i: (i, 0)),
            pl.BlockSpec((1, 32), lambda i: (0, 0)),
        ],
        out_specs=pl.BlockSpec((_R, 32), lambda i: (i, 0)),
        out_shape=jax.ShapeDtypeStruct((N, 32), jnp.bfloat16),
    )(agg3, agg3, xw2, dinv, b2.reshape(1, 32))


def _tc_mlp(efsp, Wp1, bp1, Wp2, bp2):
    EP4 = ELP // 4
    R2 = 1792
    eye4 = jnp.eye(4, dtype=jnp.float32)
    W1b = jnp.kron(eye4, Wp1)
    b1b = jnp.tile(bp1, 4).reshape(1, 64)
    W2b = jnp.kron(eye4, Wp2)
    b2b = jnp.tile(bp2, 4).reshape(1, 4)

    def body(ef_ref, w1_ref, b1_ref, w2_ref, b2_ref, o_ref):
        ef = ef_ref[...].astype(jnp.float32)
        e = jnp.dot(ef, w1_ref[...],
                    preferred_element_type=jnp.float32) + b1_ref[...]
        e = jnp.maximum(e, 0.0)
        z = jnp.dot(e, w2_ref[...],
                    preferred_element_type=jnp.float32) + b2_ref[...]
        o_ref[...] = 1.0 / (1.0 + jnp.exp(-z))
    return pl.pallas_call(
        body,
        grid=(EP4 // R2,),
        in_specs=[
            pl.BlockSpec((R2, 128), lambda i: (i, 0)),
            pl.BlockSpec((128, 64), lambda i: (0, 0)),
            pl.BlockSpec((1, 64), lambda i: (0, 0)),
            pl.BlockSpec((64, 4), lambda i: (0, 0)),
            pl.BlockSpec((1, 4), lambda i: (0, 0)),
        ],
        out_specs=pl.BlockSpec((R2, 4), lambda i: (i, 0)),
        out_shape=jax.ShapeDtypeStruct((EP4, 4), jnp.float32),
    )(efsp, W1b, b1b, W2b, b2b)


def kernel(x, edge_index, edge_label_index,
           W_enc, b_enc, W1, b1, W2, b2, Wp1, bp1, Wp2, bp2):
    f32 = jnp.float32
    i32 = jnp.int32

    src = edge_index[0]
    dst = edge_index[1]
    src_p = jnp.concatenate(
        [src, jnp.zeros((EP - E,), i32)]).reshape(EP // CHUNK, CHUNK)
    dst_p = jnp.concatenate(
        [dst, jnp.full((EP - E,), N, i32)]).reshape(EP // CHUNK, CHUNK)
    lsrc_p = jnp.concatenate(
        [edge_label_index[0], jnp.zeros((ELP - EL,), i32)]
    ).reshape(ELP // CHUNK, CHUNK)
    ldst_p = jnp.concatenate(
        [edge_label_index[1], jnp.zeros((ELP - EL,), i32)]
    ).reshape(ELP // CHUNK, CHUNK)

    z1 = jnp.zeros((APT,), f32)
    z32 = jnp.zeros((ZC, 32), f32)
    z16 = jnp.zeros((ZC, 16), f32)
    ones = jnp.ones((CHUNK,), f32)

    degp = _deg_kernel(dst_p, ones, z1)
    degp_t = degp[:, :N].T
    xw1, dinv, t3 = _tc_encoder(x, degp_t, W_enc, b_enc, W1)
    agg3 = _conv_agg(t3, src_p, dst_p, z32, 32, CHUNK, KE0, KE1, 14)
    xw2, t32 = _tc_conv1_post(agg3, xw1, dinv, b1, W2)
    src_p2 = src_p.reshape(EP // (2 * CHUNK), 2 * CHUNK)
    dst_p2 = dst_p.reshape(EP // (2 * CHUNK), 2 * CHUNK)
    agg23 = _conv_agg(t32, src_p2, dst_p2, z16, 16, 2 * CHUNK,
                      KE0 // 2, KE1 // 2, 7)
    h2 = _tc_conv2_post(agg23, xw2, dinv, b2)
    efp = _label_gather(h2, lsrc_p, ldst_p)
    out = _tc_mlp(efp, Wp1, bp1, Wp2, bp2)
    return out.reshape(ELP)[:EL]
